# Initial kernel scaffold; baseline (speedup 1.0000x reference)
import jax
import jax.numpy as jnp
from jax.experimental import pallas as pl


def kernel(xyz, points, offset, W_l0, g_l0, b_l0, W_f0, g_f0, b_f0, W1, bc1, g1, b1):
    raise NotImplementedError("write your pallas kernel here")


# speedup vs baseline: 3.5481x; 3.5481x over previous
import functools

import jax
import jax.numpy as jnp
from jax import lax
from jax.experimental import pallas as pl
from jax.experimental.pallas import tpu as pltpu
from jax.experimental.pallas import tpu_sc as plsc

B = 8
NPER = 2048
N = B * NPER
NS = 32
CIN = 6
C0 = 32
C1 = 64
NE = N * NS
EPS = 1e-5

_BQ = 256
_QB = NPER // _BQ


def _knn_body(q_ref, xt_ref, out_ref):
    seg = pl.program_id(0)
    q = q_ref[...]
    xt = xt_ref[...]
    d = (q[:, 0:1] - xt[0:1, :]) ** 2
    d = d + (q[:, 1:2] - xt[1:2, :]) ** 2
    d = d + (q[:, 2:3] - xt[2:3, :]) ** 2
    iota = lax.broadcasted_iota(jnp.int32, (_BQ, NPER), 1)
    kiota = lax.broadcasted_iota(jnp.int32, (_BQ, NS), 1)
    inf = jnp.float32(3.4e38)

    def step(k, carry):
        dc, acc = carry
        m = jnp.min(dc, axis=1, keepdims=True)
        sel = jnp.min(jnp.where(dc == m, iota, NPER), axis=1,
                      keepdims=True)
        acc = jnp.where(kiota == k, sel, acc)
        dc = jnp.where(iota == sel, inf, dc)
        return dc, acc

    _, acc = lax.fori_loop(0, NS, step,
                           (d, jnp.zeros((_BQ, NS), jnp.int32)))
    out_ref[...] = acc + seg * NPER


def _knn(xyz):
    xyzT = xyz.T
    return pl.pallas_call(
        _knn_body,
        grid=(B, _QB),
        in_specs=[
            pl.BlockSpec((_BQ, 3), lambda s, q: (s * _QB + q, 0)),
            pl.BlockSpec((3, NPER), lambda s, q: (0, s)),
        ],
        out_specs=pl.BlockSpec((_BQ, NS), lambda s, q: (s * _QB + q, 0)),
        out_shape=jax.ShapeDtypeStruct((N, NS), jnp.int32),
    )(xyz, xyzT)


_NW = 32
_RPW = NE // _NW
_CH_ROWS = 2048
_CH_Q = _CH_ROWS // NS
_NCH = _RPW // _CH_ROWS
_GPC = _CH_ROWS // 128


def _sc_gather_sub(up, vself, idx2):
    mesh = plsc.VectorSubcoreMesh(core_axis_name="c", subcore_axis_name="s")

    @functools.partial(
        pl.kernel,
        out_type=jax.ShapeDtypeStruct((NE, 16), jnp.float32),
        mesh=mesh,
        compiler_params=pltpu.CompilerParams(use_tc_tiling_on_sc=False),
        scratch_types=[
            pltpu.VMEM((_GPC, 128), jnp.int32),
            pltpu.VMEM((_CH_ROWS, 16), jnp.float32),
            pltpu.VMEM((_CH_Q, 16), jnp.float32),
            pltpu.SemaphoreType.DMA,
        ],
    )
    def body(up_hbm, vs_hbm, idx_hbm, out_hbm, idx_v, rows_v, vself_v, sem):
        wid = lax.axis_index("s") * 2 + lax.axis_index("c")
        row_base = wid * _RPW
        q_base = wid * (_RPW // NS)

        def chunk(c, _):
            rb = pl.multiple_of(row_base + c * _CH_ROWS, _CH_ROWS)
            qb = pl.multiple_of(q_base + c * _CH_Q, _CH_Q)
            ib = pl.multiple_of(rb // 128, _GPC)
            pltpu.sync_copy(idx_hbm.at[pl.ds(ib, _GPC)], idx_v)
            cps = [
                pltpu.make_async_copy(
                    up_hbm.at[idx_v.at[j]],
                    rows_v.at[pl.ds(j * 128, 128)],
                    sem,
                )
                for j in range(_GPC)
            ]
            for cp in cps:
                cp.start()
            for cp in cps:
                cp.wait()
            pltpu.sync_copy(vs_hbm.at[pl.ds(qb, _CH_Q)], vself_v)

            def subq(i, _):
                v = vself_v[i]
                base = i * NS
                for s2 in range(NS):
                    rows_v[base + s2] = rows_v[base + s2] - v
                return 0

            lax.fori_loop(0, _CH_Q, subq, 0)
            pltpu.sync_copy(rows_v, out_hbm.at[pl.ds(rb, _CH_ROWS)])
            return 0

        lax.fori_loop(0, _NCH, chunk, 0)

    return body(up, vself, idx2)


_BR3 = 8192


def _mom_body(gu_ref, m_ref):
    @pl.when(pl.program_id(0) == 0)
    def _():
        m_ref[...] = jnp.zeros_like(m_ref)

    g = gu_ref[...]
    m_ref[...] += lax.dot_general(g, g, (((0,), (0,)), ((), ())),
                                  preferred_element_type=jnp.float32)


def _moments(gu):
    return pl.pallas_call(
        _mom_body,
        grid=(NE // _BR3,),
        in_specs=[pl.BlockSpec((_BR3, 16), lambda i: (i, 0))],
        out_specs=pl.BlockSpec((16, 16), lambda i: (0, 0)),
        out_shape=jax.ShapeDtypeStruct((16, 16), jnp.float32),
    )(gu)


_BR4 = 8192


def _hmom_body(gu_ref, t_ref, mh_ref, sg_ref):
    @pl.when(pl.program_id(0) == 0)
    def _():
        mh_ref[...] = jnp.zeros_like(mh_ref)
        sg_ref[...] = jnp.zeros_like(sg_ref)

    g = gu_ref[...]
    t = t_ref[...]
    h = jnp.maximum(
        lax.dot_general(g, t, (((1,), (1,)), ((), ())),
                        preferred_element_type=jnp.float32), 0.0)
    mh_ref[...] += lax.dot_general(h, h, (((0,), (0,)), ((), ())),
                                   preferred_element_type=jnp.float32)
    sg_ref[...] += lax.dot_general(h, g, (((0,), (0,)), ((), ())),
                                   preferred_element_type=jnp.float32)


def _hmoments(gu, t):
    return pl.pallas_call(
        _hmom_body,
        grid=(NE // _BR4,),
        in_specs=[
            pl.BlockSpec((_BR4, 16), lambda i: (i, 0)),
            pl.BlockSpec((C0, 16), lambda i: (0, 0)),
        ],
        out_specs=[
            pl.BlockSpec((C0, C0), lambda i: (0, 0)),
            pl.BlockSpec((C0, 16), lambda i: (0, 0)),
        ],
        out_shape=[
            jax.ShapeDtypeStruct((C0, C0), jnp.float32),
            jax.ShapeDtypeStruct((C0, 16), jnp.float32),
        ],
    )(gu, t)


_BQ5 = 64
_BR5 = _BQ5 * NS


def _final_body(gu_ref, t_ref, w_ref, out_ref):
    g = gu_ref[...]
    t = t_ref[...]
    w = w_ref[...]
    h = jnp.maximum(
        lax.dot_general(g, t, (((1,), (1,)), ((), ())),
                        preferred_element_type=jnp.float32), 0.0)
    hg = jnp.concatenate([h, g], axis=1)
    y = jnp.maximum(
        lax.dot_general(hg, w, (((1,), (1,)), ((), ())),
                        preferred_element_type=jnp.float32), 0.0)
    out_ref[...] = jnp.max(y.reshape(_BQ5, NS, C1), axis=1)


def _final(gu, t, w):
    return pl.pallas_call(
        _final_body,
        grid=(NE // _BR5,),
        in_specs=[
            pl.BlockSpec((_BR5, 16), lambda i: (i, 0)),
            pl.BlockSpec((C0, 16), lambda i: (0, 0)),
            pl.BlockSpec((C1, C0 + 16), lambda i: (0, 0)),
        ],
        out_specs=pl.BlockSpec((_BQ5, C1), lambda i: (i, 0)),
        out_shape=jax.ShapeDtypeStruct((N, C1), jnp.float32),
    )(gu, t, w)


def kernel(xyz, points, offset, W_l0, g_l0, b_l0, W_f0, g_f0, b_f0,
           W1, bc1, g1, b1):
    gidx = _knn(xyz)

    pad = jnp.zeros((N, 16 - 3 - CIN), jnp.float32)
    up = jnp.concatenate(
        [xyz, points, pad[:, :-1], jnp.ones((N, 1), jnp.float32)], axis=1)
    vself = jnp.concatenate([xyz, jnp.zeros((N, 13), jnp.float32)], axis=1)
    idx2 = gidx.reshape(NE // 128, 128)

    gu = _sc_gather_sub(up, vself, idx2)

    m = _moments(gu)
    e = jnp.float32(NE)
    mean_gx = m[0:3, 15] / e
    cov_gx = m[0:3, 0:3] / e - jnp.outer(mean_gx, mean_gx)
    mu_l = W_l0 @ mean_gx
    var_l = jnp.einsum('oc,cd,od->o', W_l0, cov_gx, W_l0)
    s_l = g_l0 / jnp.sqrt(var_l + EPS)
    mean_p = m[3:3 + CIN, 15] / e
    cov_p = m[3:3 + CIN, 3:3 + CIN] / e - jnp.outer(mean_p, mean_p)
    mu_f = W_f0 @ mean_p
    var_f = jnp.einsum('oc,cd,od->o', W_f0, cov_p, W_f0)
    s_f = g_f0 / jnp.sqrt(var_f + EPS)
    t = jnp.zeros((C0, 16), jnp.float32)
    t = t.at[:, 0:3].set(s_l[:, None] * W_l0)
    t = t.at[:, 3:3 + CIN].set(s_f[:, None] * W_f0)
    t = t.at[:, 15].set(b_l0 - s_l * mu_l + b_f0 - s_f * mu_f)

    mh, sg = _hmoments(gu, t)
    sh = sg[:, 15]
    w1sh = W1 @ sh / e
    mean1 = w1sh + bc1
    ey2 = jnp.einsum('oc,cd,od->o', W1, mh, W1) / e + 2 * bc1 * w1sh + bc1 ** 2
    var1 = ey2 - mean1 ** 2
    s1 = g1 / jnp.sqrt(var1 + EPS)
    wfull = jnp.zeros((C1, C0 + 16), jnp.float32)
    wfull = wfull.at[:, 0:C0].set(s1[:, None] * W1)
    wfull = wfull.at[:, C0 + 15].set(b1 + s1 * (bc1 - mean1))

    new_feats = _final(gu, t, wfull)
    return (xyz, new_feats, offset)


# speedup vs baseline: 4.7740x; 1.3455x over previous
import functools

import jax
import jax.numpy as jnp
from jax import lax
from jax.experimental import pallas as pl
from jax.experimental.pallas import tpu as pltpu
from jax.experimental.pallas import tpu_sc as plsc

B = 8
NPER = 2048
N = B * NPER
NS = 32
CIN = 6
C0 = 32
C1 = 64
NE = N * NS
EPS = 1e-5

_BQ = 256
_QB = NPER // _BQ


def _knn_body(q_ref, xt_ref, out_ref):
    seg = pl.program_id(0)
    q = q_ref[...]
    xt = xt_ref[...]
    d = (q[:, 0:1] - xt[0:1, :]) ** 2
    d = d + (q[:, 1:2] - xt[1:2, :]) ** 2
    d = d + (q[:, 2:3] - xt[2:3, :]) ** 2
    fiota = lax.broadcasted_iota(jnp.int32, (_BQ, NPER), 1).astype(jnp.float32)
    kiota = lax.broadcasted_iota(jnp.int32, (_BQ, NS), 1)
    inf = jnp.float32(3.4e38)
    fnper = jnp.float32(NPER)

    def step(k, carry):
        dc, acc = carry
        m = jnp.min(dc, axis=1, keepdims=True)
        sel = jnp.min(jnp.where(dc == m, fiota, fnper), axis=1,
                      keepdims=True)
        acc = jnp.where(kiota == k, sel.astype(jnp.int32), acc)
        dc = jnp.where(fiota == sel, inf, dc)
        return dc, acc

    _, acc = lax.fori_loop(0, NS, step,
                           (d, jnp.zeros((_BQ, NS), jnp.int32)))
    out_ref[...] = acc + seg * NPER


def _knn_fallback(xyz, xyzT):
    return pl.pallas_call(
        _knn_body,
        grid=(B, _QB),
        in_specs=[
            pl.BlockSpec((_BQ, 3), lambda s, q: (s * _QB + q, 0)),
            pl.BlockSpec((3, NPER), lambda s, q: (0, s)),
        ],
        out_specs=pl.BlockSpec((_BQ, NS), lambda s, q: (s * _QB + q, 0)),
        out_shape=jax.ShapeDtypeStruct((N, NS), jnp.int32),
    )(xyz, xyzT)


def _dist_body(q_ref, xt_ref, d_ref, m_ref):
    q = q_ref[...]
    xt = xt_ref[...]
    d = (q[:, 0:1] - xt[0:1, :]) ** 2
    d = d + (q[:, 1:2] - xt[1:2, :]) ** 2
    d = d + (q[:, 2:3] - xt[2:3, :]) ** 2
    m = d[:, 0:128]
    for c in range(1, NPER // 128):
        m = jnp.minimum(m, d[:, c * 128:(c + 1) * 128])
    d_ref[...] = d
    m_ref[...] = m


def _dist(xyz, xyzT):
    return pl.pallas_call(
        _dist_body,
        grid=(B, _QB),
        in_specs=[
            pl.BlockSpec((_BQ, 3), lambda s, q: (s * _QB + q, 0)),
            pl.BlockSpec((3, NPER), lambda s, q: (0, s)),
        ],
        out_specs=[
            pl.BlockSpec((_BQ, NPER), lambda s, q: (s * _QB + q, 0)),
            pl.BlockSpec((_BQ, 128), lambda s, q: (s * _QB + q, 0)),
        ],
        out_shape=[
            jax.ShapeDtypeStruct((N, NPER), jnp.float32),
            jax.ShapeDtypeStruct((N, 128), jnp.float32),
        ],
    )(xyz, xyzT)


_INF = 3.4e38
_NW = 32
_RPW1 = N // _NW
_DCH = 8
_MCH = 16


def _mergek16(a, b):
    rb = lax.rev(b, (0,))
    s = jnp.minimum(a, rb)
    t = jnp.maximum(a, rb)
    return lax.sort(s), lax.sort(t)


def _mergek32(a0, a1, b0, b1):
    rb0 = lax.rev(b0, (0,))
    rb1 = lax.rev(b1, (0,))
    s0 = jnp.minimum(a0, rb1)
    s1 = jnp.minimum(a1, rb0)
    u = jnp.minimum(s0, s1)
    v = jnp.maximum(s0, s1)
    return lax.sort(u), lax.sort(v)


def _sc_topk(darr, m128):
    mesh = plsc.VectorSubcoreMesh(core_axis_name="c", subcore_axis_name="s")

    @functools.partial(
        pl.kernel,
        out_type=[
            jax.ShapeDtypeStruct((N, NS), jnp.int32),
            jax.ShapeDtypeStruct((_NW, 16), jnp.int32),
        ],
        mesh=mesh,
        compiler_params=pltpu.CompilerParams(use_tc_tiling_on_sc=False,
                                             needs_layout_passes=False),
        scratch_types=[
            pltpu.VMEM((2, _DCH, NPER), jnp.float32),
            pltpu.VMEM((_MCH, 128), jnp.float32),
            pltpu.VMEM((NPER,), jnp.int32),
            pltpu.VMEM((128,), jnp.float32),
            pltpu.VMEM((128,), jnp.int32),
            pltpu.VMEM((2, _MCH, NS), jnp.int32),
            pltpu.VMEM((NS,), jnp.int32),
            pltpu.VMEM((16,), jnp.int32),
            pltpu.SemaphoreType.DMA,
            pltpu.SemaphoreType.DMA,
        ],
    )
    def body(d_hbm, m_hbm, gidx_hbm, ovf_hbm, dbuf, mbuf, ramp, cbuf, ibuf,
             ostage, obuf32, obuf, dsem, osem):
        wid = lax.axis_index("s") * 2 + lax.axis_index("c")
        row0 = wid * _RPW1
        segbase = (row0 // NPER) * NPER
        iota16 = lax.iota(jnp.int32, 16)

        def mkramp(j, _):
            ramp[pl.ds(j * 16, 16)] = iota16 + (segbase + j * 16)
            return 0
        lax.fori_loop(0, NPER // 16, mkramp, 0)

        def dcopy(c, buf):
            rb = pl.multiple_of(row0 + c * _DCH, _DCH)
            return pltpu.make_async_copy(
                d_hbm.at[pl.ds(rb, _DCH)], dbuf.at[buf], dsem)

        def ocopy(bi):
            orb = pl.multiple_of(row0 + bi * _MCH, _MCH)
            return pltpu.make_async_copy(
                ostage.at[bi % 2], gidx_hbm.at[pl.ds(orb, _MCH)], osem)

        dcopy(0, 0).start()

        def do_row(dch, rl, rloc, bi):
            g = [lax.sort(mbuf[rloc, pl.ds(gg * 16, 16)]) for gg in range(8)]
            p0 = _mergek16(g[0], g[1])
            p1 = _mergek16(g[2], g[3])
            p2 = _mergek16(g[4], g[5])
            p3 = _mergek16(g[6], g[7])
            q0 = _mergek32(*p0, *p1)
            q1 = _mergek32(*p2, *p3)
            _, f1 = _mergek32(*q0, *q1)
            t2 = jnp.max(f1)

            for gg in range(8):
                cbuf[pl.ds(gg * 16, 16)] = jnp.full((16,), _INF,
                                                    dtype=jnp.float32)

            def comp(jb, offv):
                for jj in range(4):
                    j = jb * 4 + jj
                    v = dbuf[dch, rl, pl.ds(j * 16, 16)]
                    msk = v <= t2
                    cs = jnp.cumsum(msk.astype(jnp.int32))
                    pos = jnp.minimum(offv + cs, 127)
                    plsc.store_scatter(cbuf, [pos], v, mask=msk)
                    plsc.store_scatter(ibuf, [pos],
                                       ramp[pl.ds(j * 16, 16)], mask=msk)
                    offv = offv + plsc.all_reduce_population_count(msk)
                return offv

            offv = lax.fori_loop(0, (NPER // 16) // 4, comp,
                                 jnp.full((16,), -1, jnp.int32))

            g2 = [lax.sort(cbuf[pl.ds(gg * 16, 16)]) for gg in range(8)]
            r0 = _mergek16(g2[0], g2[1])
            r1 = _mergek16(g2[2], g2[3])
            r2 = _mergek16(g2[4], g2[5])
            r3 = _mergek16(g2[6], g2[7])
            w0 = _mergek32(*r0, *r1)
            w1 = _mergek32(*r2, *r3)
            _, s1 = _mergek32(*w0, *w1)
            t32 = jnp.max(s1)

            off2 = jnp.full((16,), -1, jnp.int32)
            for gg in range(8):
                v = cbuf[pl.ds(gg * 16, 16)]
                msk2 = v <= t32
                cs2 = jnp.cumsum(msk2.astype(jnp.int32))
                pos2 = off2 + cs2
                msk3 = msk2 & (pos2 < NS)
                plsc.store_scatter(obuf32, [jnp.minimum(pos2, NS - 1)],
                                   ibuf[pl.ds(gg * 16, 16)], mask=msk3)
                off2 = off2 + plsc.all_reduce_population_count(msk2)
            ostage[bi % 2, rloc, pl.ds(0, 16)] = obuf32[pl.ds(0, 16)]
            ostage[bi % 2, rloc, pl.ds(16, 16)] = obuf32[pl.ds(16, 16)]
            return offv + 1

        def blk16(bi, ofmax):
            @pl.when(bi >= 2)
            def _():
                ocopy(0).wait()
            mrb = pl.multiple_of(row0 + bi * _MCH, _MCH)
            pltpu.sync_copy(m_hbm.at[pl.ds(mrb, _MCH)], mbuf)
            for rb4 in range(_MCH // _DCH):
                c = bi * (_MCH // _DCH) + rb4
                dcopy(c, c % 2).wait()

                @pl.when(c + 1 < _RPW1 // _DCH)
                def _():
                    dcopy(c + 1, (c + 1) % 2).start()
                for rl in range(_DCH):
                    cnt = do_row(c % 2, rl, rb4 * _DCH + rl, bi)
                    ofmax = jnp.maximum(ofmax, cnt)
            ocopy(bi).start()
            return ofmax

        ofmax = lax.fori_loop(0, _RPW1 // _MCH, blk16,
                              jnp.zeros((16,), jnp.int32))
        ocopy(0).wait()
        ocopy(0).wait()
        obuf[pl.ds(0, 16)] = ofmax
        pltpu.sync_copy(obuf, ovf_hbm.at[wid])

    return body(darr, m128)


_RPW = NE // _NW
_CH_ROWS = 2048
_CH_Q = _CH_ROWS // NS
_NCH = _RPW // _CH_ROWS
_GPC = _CH_ROWS // 128


def _sc_gather_sub(up, vself, idx2):
    mesh = plsc.VectorSubcoreMesh(core_axis_name="c", subcore_axis_name="s")

    @functools.partial(
        pl.kernel,
        out_type=jax.ShapeDtypeStruct((NE, 16), jnp.float32),
        mesh=mesh,
        compiler_params=pltpu.CompilerParams(use_tc_tiling_on_sc=False),
        scratch_types=[
            pltpu.VMEM((_GPC, 128), jnp.int32),
            pltpu.VMEM((_CH_ROWS, 16), jnp.float32),
            pltpu.VMEM((_CH_Q, 16), jnp.float32),
            pltpu.SemaphoreType.DMA,
        ],
    )
    def body(up_hbm, vs_hbm, idx_hbm, out_hbm, idx_v, rows_v, vself_v, sem):
        wid = lax.axis_index("s") * 2 + lax.axis_index("c")
        row_base = wid * _RPW
        q_base = wid * (_RPW // NS)

        def chunk(c, _):
            rb = pl.multiple_of(row_base + c * _CH_ROWS, _CH_ROWS)
            qb = pl.multiple_of(q_base + c * _CH_Q, _CH_Q)
            ib = pl.multiple_of(rb // 128, _GPC)
            pltpu.sync_copy(idx_hbm.at[pl.ds(ib, _GPC)], idx_v)
            cps = [
                pltpu.make_async_copy(
                    up_hbm.at[idx_v.at[j]],
                    rows_v.at[pl.ds(j * 128, 128)],
                    sem,
                )
                for j in range(_GPC)
            ]
            for cp in cps:
                cp.start()
            for cp in cps:
                cp.wait()
            pltpu.sync_copy(vs_hbm.at[pl.ds(qb, _CH_Q)], vself_v)

            def subq(i, _):
                v = vself_v[i]
                base = i * NS
                for s2 in range(NS):
                    rows_v[base + s2] = rows_v[base + s2] - v
                return 0

            lax.fori_loop(0, _CH_Q, subq, 0)
            pltpu.sync_copy(rows_v, out_hbm.at[pl.ds(rb, _CH_ROWS)])
            return 0

        lax.fori_loop(0, _NCH, chunk, 0)

    return body(up, vself, idx2)


_BR3 = 8192


def _mom_body(gu_ref, m_ref):
    @pl.when(pl.program_id(0) == 0)
    def _():
        m_ref[...] = jnp.zeros_like(m_ref)

    g = gu_ref[...]
    m_ref[...] += lax.dot_general(g, g, (((0,), (0,)), ((), ())),
                                  preferred_element_type=jnp.float32)


def _moments(gu):
    return pl.pallas_call(
        _mom_body,
        grid=(NE // _BR3,),
        in_specs=[pl.BlockSpec((_BR3, 16), lambda i: (i, 0))],
        out_specs=pl.BlockSpec((16, 16), lambda i: (0, 0)),
        out_shape=jax.ShapeDtypeStruct((16, 16), jnp.float32),
    )(gu)


_BR4 = 8192


def _hmom_body(gu_ref, t_ref, mh_ref, sg_ref):
    @pl.when(pl.program_id(0) == 0)
    def _():
        mh_ref[...] = jnp.zeros_like(mh_ref)
        sg_ref[...] = jnp.zeros_like(sg_ref)

    g = gu_ref[...]
    t = t_ref[...]
    h = jnp.maximum(
        lax.dot_general(g, t, (((1,), (1,)), ((), ())),
                        preferred_element_type=jnp.float32), 0.0)
    mh_ref[...] += lax.dot_general(h, h, (((0,), (0,)), ((), ())),
                                   preferred_element_type=jnp.float32)
    sg_ref[...] += lax.dot_general(h, g, (((0,), (0,)), ((), ())),
                                   preferred_element_type=jnp.float32)


def _hmoments(gu, t):
    return pl.pallas_call(
        _hmom_body,
        grid=(NE // _BR4,),
        in_specs=[
            pl.BlockSpec((_BR4, 16), lambda i: (i, 0)),
            pl.BlockSpec((C0, 16), lambda i: (0, 0)),
        ],
        out_specs=[
            pl.BlockSpec((C0, C0), lambda i: (0, 0)),
            pl.BlockSpec((C0, 16), lambda i: (0, 0)),
        ],
        out_shape=[
            jax.ShapeDtypeStruct((C0, C0), jnp.float32),
            jax.ShapeDtypeStruct((C0, 16), jnp.float32),
        ],
    )(gu, t)


_BQ5 = 64
_BR5 = _BQ5 * NS


def _final_body(gu_ref, t_ref, w_ref, out_ref):
    g = gu_ref[...]
    t = t_ref[...]
    w = w_ref[...]
    h = jnp.maximum(
        lax.dot_general(g, t, (((1,), (1,)), ((), ())),
                        preferred_element_type=jnp.float32), 0.0)
    hg = jnp.concatenate([h, g], axis=1)
    y = jnp.maximum(
        lax.dot_general(hg, w, (((1,), (1,)), ((), ())),
                        preferred_element_type=jnp.float32), 0.0)
    out_ref[...] = jnp.max(y.reshape(_BQ5, NS, C1), axis=1)


def _final(gu, t, w):
    return pl.pallas_call(
        _final_body,
        grid=(NE // _BR5,),
        in_specs=[
            pl.BlockSpec((_BR5, 16), lambda i: (i, 0)),
            pl.BlockSpec((C0, 16), lambda i: (0, 0)),
            pl.BlockSpec((C1, C0 + 16), lambda i: (0, 0)),
        ],
        out_specs=pl.BlockSpec((_BQ5, C1), lambda i: (i, 0)),
        out_shape=jax.ShapeDtypeStruct((N, C1), jnp.float32),
    )(gu, t, w)


def kernel(xyz, points, offset, W_l0, g_l0, b_l0, W_f0, g_f0, b_f0,
           W1, bc1, g1, b1):
    xyzT = xyz.T
    darr, m128 = _dist(xyz, xyzT)
    gidx_fast, ovf = _sc_topk(darr, m128)
    gidx = lax.cond(jnp.max(ovf) > 128,
                    lambda: _knn_fallback(xyz, xyzT),
                    lambda: gidx_fast)

    pad = jnp.zeros((N, 16 - 3 - CIN), jnp.float32)
    up = jnp.concatenate(
        [xyz, points, pad[:, :-1], jnp.ones((N, 1), jnp.float32)], axis=1)
    vself = jnp.concatenate([xyz, jnp.zeros((N, 13), jnp.float32)], axis=1)
    idx2 = gidx.reshape(NE // 128, 128)

    gu = _sc_gather_sub(up, vself, idx2)

    m = _moments(gu)
    e = jnp.float32(NE)
    mean_gx = m[0:3, 15] / e
    cov_gx = m[0:3, 0:3] / e - jnp.outer(mean_gx, mean_gx)
    mu_l = W_l0 @ mean_gx
    var_l = jnp.einsum('oc,cd,od->o', W_l0, cov_gx, W_l0)
    s_l = g_l0 / jnp.sqrt(var_l + EPS)
    mean_p = m[3:3 + CIN, 15] / e
    cov_p = m[3:3 + CIN, 3:3 + CIN] / e - jnp.outer(mean_p, mean_p)
    mu_f = W_f0 @ mean_p
    var_f = jnp.einsum('oc,cd,od->o', W_f0, cov_p, W_f0)
    s_f = g_f0 / jnp.sqrt(var_f + EPS)
    t = jnp.zeros((C0, 16), jnp.float32)
    t = t.at[:, 0:3].set(s_l[:, None] * W_l0)
    t = t.at[:, 3:3 + CIN].set(s_f[:, None] * W_f0)
    t = t.at[:, 15].set(b_l0 - s_l * mu_l + b_f0 - s_f * mu_f)

    mh, sg = _hmoments(gu, t)
    sh = sg[:, 15]
    w1sh = W1 @ sh / e
    mean1 = w1sh + bc1
    ey2 = jnp.einsum('oc,cd,od->o', W1, mh, W1) / e + 2 * bc1 * w1sh + bc1 ** 2
    var1 = ey2 - mean1 ** 2
    s1 = g1 / jnp.sqrt(var1 + EPS)
    wfull = jnp.zeros((C1, C0 + 16), jnp.float32)
    wfull = wfull.at[:, 0:C0].set(s1[:, None] * W1)
    wfull = wfull.at[:, C0 + 15].set(b1 + s1 * (bc1 - mean1))

    new_feats = _final(gu, t, wfull)
    return (xyz, new_feats, offset)


# speedup vs baseline: 5.0575x; 1.0594x over previous
import functools

import jax
import jax.numpy as jnp
from jax import lax
from jax.experimental import pallas as pl
from jax.experimental.pallas import tpu as pltpu
from jax.experimental.pallas import tpu_sc as plsc

B = 8
NPER = 2048
N = B * NPER
NS = 32
CIN = 6
C0 = 32
C1 = 64
NE = N * NS
EPS = 1e-5

_BQ = 256
_QB = NPER // _BQ


def _knn_body(q_ref, xt_ref, out_ref):
    seg = pl.program_id(0)
    q = q_ref[...]
    xt = xt_ref[...]
    d = (q[:, 0:1] - xt[0:1, :]) ** 2
    d = d + (q[:, 1:2] - xt[1:2, :]) ** 2
    d = d + (q[:, 2:3] - xt[2:3, :]) ** 2
    fiota = lax.broadcasted_iota(jnp.int32, (_BQ, NPER), 1).astype(jnp.float32)
    kiota = lax.broadcasted_iota(jnp.int32, (_BQ, NS), 1)
    inf = jnp.float32(3.4e38)
    fnper = jnp.float32(NPER)

    def step(k, carry):
        dc, acc = carry
        m = jnp.min(dc, axis=1, keepdims=True)
        sel = jnp.min(jnp.where(dc == m, fiota, fnper), axis=1,
                      keepdims=True)
        acc = jnp.where(kiota == k, sel.astype(jnp.int32), acc)
        dc = jnp.where(fiota == sel, inf, dc)
        return dc, acc

    _, acc = lax.fori_loop(0, NS, step,
                           (d, jnp.zeros((_BQ, NS), jnp.int32)))
    out_ref[...] = acc + seg * NPER


def _knn_fallback(xyz, xyzT):
    return pl.pallas_call(
        _knn_body,
        grid=(B, _QB),
        in_specs=[
            pl.BlockSpec((_BQ, 3), lambda s, q: (s * _QB + q, 0)),
            pl.BlockSpec((3, NPER), lambda s, q: (0, s)),
        ],
        out_specs=pl.BlockSpec((_BQ, NS), lambda s, q: (s * _QB + q, 0)),
        out_shape=jax.ShapeDtypeStruct((N, NS), jnp.int32),
    )(xyz, xyzT)


def _dist_body(q_ref, xt_ref, d_ref, m_ref):
    q = q_ref[...]
    xt = xt_ref[...]
    d = (q[:, 0:1] - xt[0:1, :]) ** 2
    d = d + (q[:, 1:2] - xt[1:2, :]) ** 2
    d = d + (q[:, 2:3] - xt[2:3, :]) ** 2
    m = d[:, 0:128]
    for c in range(1, NPER // 128):
        m = jnp.minimum(m, d[:, c * 128:(c + 1) * 128])
    d_ref[...] = d
    m_ref[...] = m


def _dist(xyz, xyzT):
    return pl.pallas_call(
        _dist_body,
        grid=(B, _QB),
        in_specs=[
            pl.BlockSpec((_BQ, 3), lambda s, q: (s * _QB + q, 0)),
            pl.BlockSpec((3, NPER), lambda s, q: (0, s)),
        ],
        out_specs=[
            pl.BlockSpec((_BQ, NPER), lambda s, q: (s * _QB + q, 0)),
            pl.BlockSpec((_BQ, 128), lambda s, q: (s * _QB + q, 0)),
        ],
        out_shape=[
            jax.ShapeDtypeStruct((N, NPER), jnp.float32),
            jax.ShapeDtypeStruct((N, 128), jnp.float32),
        ],
    )(xyz, xyzT)


_INF = 3.4e38
_NW = 32
_RPW1 = N // _NW
_DCH = 8
_MCH = 16


def _mergek16(a, b):
    rb = lax.rev(b, (0,))
    s = jnp.minimum(a, rb)
    t = jnp.maximum(a, rb)
    return lax.sort(s), lax.sort(t)


def _mergek32(a0, a1, b0, b1):
    rb0 = lax.rev(b0, (0,))
    rb1 = lax.rev(b1, (0,))
    s0 = jnp.minimum(a0, rb1)
    s1 = jnp.minimum(a1, rb0)
    u = jnp.minimum(s0, s1)
    v = jnp.maximum(s0, s1)
    return lax.sort(u), lax.sort(v)


def _sc_topk(darr, m128):
    mesh = plsc.VectorSubcoreMesh(core_axis_name="c", subcore_axis_name="s")

    @functools.partial(
        pl.kernel,
        out_type=[
            jax.ShapeDtypeStruct((N, NS), jnp.int32),
            jax.ShapeDtypeStruct((_NW, 16), jnp.int32),
        ],
        mesh=mesh,
        compiler_params=pltpu.CompilerParams(use_tc_tiling_on_sc=False,
                                             needs_layout_passes=False),
        scratch_types=[
            pltpu.VMEM((2, _DCH, NPER), jnp.float32),
            pltpu.VMEM((_MCH, 128), jnp.float32),
            pltpu.VMEM((NPER,), jnp.int32),
            pltpu.VMEM((128,), jnp.float32),
            pltpu.VMEM((128,), jnp.int32),
            pltpu.VMEM((2, _MCH, NS), jnp.int32),
            pltpu.VMEM((NS,), jnp.int32),
            pltpu.VMEM((16,), jnp.int32),
            pltpu.SemaphoreType.DMA,
            pltpu.SemaphoreType.DMA,
        ],
    )
    def body(d_hbm, m_hbm, gidx_hbm, ovf_hbm, dbuf, mbuf, ramp, cbuf, ibuf,
             ostage, obuf32, obuf, dsem, osem):
        wid = lax.axis_index("s") * 2 + lax.axis_index("c")
        row0 = wid * _RPW1
        segbase = (row0 // NPER) * NPER
        iota16 = lax.iota(jnp.int32, 16)

        def mkramp(j, _):
            ramp[pl.ds(j * 16, 16)] = iota16 + (segbase + j * 16)
            return 0
        lax.fori_loop(0, NPER // 16, mkramp, 0)

        def dcopy(c, buf):
            rb = pl.multiple_of(row0 + c * _DCH, _DCH)
            return pltpu.make_async_copy(
                d_hbm.at[pl.ds(rb, _DCH)], dbuf.at[buf], dsem)

        def ocopy(bi):
            orb = pl.multiple_of(row0 + bi * _MCH, _MCH)
            return pltpu.make_async_copy(
                ostage.at[bi % 2], gidx_hbm.at[pl.ds(orb, _MCH)], osem)

        dcopy(0, 0).start()

        def do_row(dch, rl, rloc, bi):
            g = [lax.sort(mbuf[rloc, pl.ds(gg * 16, 16)]) for gg in range(8)]
            p0 = _mergek16(g[0], g[1])
            p1 = _mergek16(g[2], g[3])
            p2 = _mergek16(g[4], g[5])
            p3 = _mergek16(g[6], g[7])
            q0 = _mergek32(*p0, *p1)
            q1 = _mergek32(*p2, *p3)
            _, f1 = _mergek32(*q0, *q1)
            t2 = jnp.max(f1)

            for gg in range(8):
                cbuf[pl.ds(gg * 16, 16)] = jnp.full((16,), _INF,
                                                    dtype=jnp.float32)

            def comp(jb, offv):
                for jj in range(4):
                    j = jb * 4 + jj
                    v = dbuf[dch, rl, pl.ds(j * 16, 16)]
                    msk = v <= t2
                    cs = jnp.cumsum(msk.astype(jnp.int32))
                    pos = jnp.minimum(offv + cs, 127)
                    plsc.store_scatter(cbuf, [pos], v, mask=msk)
                    plsc.store_scatter(ibuf, [pos],
                                       ramp[pl.ds(j * 16, 16)], mask=msk)
                    offv = offv + plsc.all_reduce_population_count(msk)
                return offv

            offv = lax.fori_loop(0, (NPER // 16) // 4, comp,
                                 jnp.full((16,), -1, jnp.int32))

            g2 = [lax.sort(cbuf[pl.ds(gg * 16, 16)]) for gg in range(8)]
            r0 = _mergek16(g2[0], g2[1])
            r1 = _mergek16(g2[2], g2[3])
            r2 = _mergek16(g2[4], g2[5])
            r3 = _mergek16(g2[6], g2[7])
            w0 = _mergek32(*r0, *r1)
            w1 = _mergek32(*r2, *r3)
            _, s1 = _mergek32(*w0, *w1)
            t32 = jnp.max(s1)

            off2 = jnp.full((16,), -1, jnp.int32)
            for gg in range(8):
                v = cbuf[pl.ds(gg * 16, 16)]
                msk2 = v <= t32
                cs2 = jnp.cumsum(msk2.astype(jnp.int32))
                pos2 = off2 + cs2
                msk3 = msk2 & (pos2 < NS)
                plsc.store_scatter(obuf32, [jnp.minimum(pos2, NS - 1)],
                                   ibuf[pl.ds(gg * 16, 16)], mask=msk3)
                off2 = off2 + plsc.all_reduce_population_count(msk2)
            ostage[bi % 2, rloc, pl.ds(0, 16)] = obuf32[pl.ds(0, 16)]
            ostage[bi % 2, rloc, pl.ds(16, 16)] = obuf32[pl.ds(16, 16)]
            return offv + 1

        def blk16(bi, ofmax):
            @pl.when(bi >= 2)
            def _():
                ocopy(0).wait()
            mrb = pl.multiple_of(row0 + bi * _MCH, _MCH)
            pltpu.sync_copy(m_hbm.at[pl.ds(mrb, _MCH)], mbuf)
            for rb4 in range(_MCH // _DCH):
                c = bi * (_MCH // _DCH) + rb4
                dcopy(c, c % 2).wait()

                @pl.when(c + 1 < _RPW1 // _DCH)
                def _():
                    dcopy(c + 1, (c + 1) % 2).start()
                for rl in range(_DCH):
                    cnt = do_row(c % 2, rl, rb4 * _DCH + rl, bi)
                    ofmax = jnp.maximum(ofmax, cnt)
            ocopy(bi).start()
            return ofmax

        ofmax = lax.fori_loop(0, _RPW1 // _MCH, blk16,
                              jnp.zeros((16,), jnp.int32))
        ocopy(0).wait()
        ocopy(0).wait()
        obuf[pl.ds(0, 16)] = ofmax
        pltpu.sync_copy(obuf, ovf_hbm.at[wid])

    return body(darr, m128)


_RPW = NE // _NW
_CH_ROWS = 2048
_CH_Q = _CH_ROWS // NS
_NCH = _RPW // _CH_ROWS
_GPC = _CH_ROWS // 128


def _sc_gather_sub(up, vself, idx2):
    mesh = plsc.VectorSubcoreMesh(core_axis_name="c", subcore_axis_name="s")

    @functools.partial(
        pl.kernel,
        out_type=jax.ShapeDtypeStruct((NE, 16), jnp.float32),
        mesh=mesh,
        compiler_params=pltpu.CompilerParams(use_tc_tiling_on_sc=False),
        scratch_types=[
            pltpu.VMEM((_GPC, 128), jnp.int32),
            pltpu.VMEM((_CH_ROWS, 16), jnp.float32),
            pltpu.VMEM((_CH_Q, 16), jnp.float32),
            pltpu.SemaphoreType.DMA,
        ],
    )
    def body(up_hbm, vs_hbm, idx_hbm, out_hbm, idx_v, rows_v, vself_v, sem):
        wid = lax.axis_index("s") * 2 + lax.axis_index("c")
        row_base = wid * _RPW
        q_base = wid * (_RPW // NS)

        def chunk(c, _):
            rb = pl.multiple_of(row_base + c * _CH_ROWS, _CH_ROWS)
            qb = pl.multiple_of(q_base + c * _CH_Q, _CH_Q)
            ib = pl.multiple_of(rb // 128, _GPC)
            pltpu.sync_copy(idx_hbm.at[pl.ds(ib, _GPC)], idx_v)
            cps = [
                pltpu.make_async_copy(
                    up_hbm.at[idx_v.at[j]],
                    rows_v.at[pl.ds(j * 128, 128)],
                    sem,
                )
                for j in range(_GPC)
            ]
            for cp in cps:
                cp.start()
            for cp in cps:
                cp.wait()
            pltpu.sync_copy(vs_hbm.at[pl.ds(qb, _CH_Q)], vself_v)

            def subq(i, _):
                v = vself_v[i]
                base = i * NS
                for s2 in range(NS):
                    rows_v[base + s2] = rows_v[base + s2] - v
                return 0

            lax.fori_loop(0, _CH_Q, subq, 0)
            pltpu.sync_copy(rows_v, out_hbm.at[pl.ds(rb, _CH_ROWS)])
            return 0

        lax.fori_loop(0, _NCH, chunk, 0)

    return body(up, vself, idx2)


_BRF = 8192
_NBF = NE // _BRF
_QF = _BRF // NS


def _fused_body(gu_ref, wl_ref, wf_ref, w1_ref, gl_ref, bl_ref, gf_ref,
                bf_ref, bc1_ref, g1_ref, b1_ref, out_ref,
                macc, mh, sg, tbuf, wbuf):
    p = pl.program_id(0)
    i = pl.program_id(1)
    e = jnp.float32(NE)

    @pl.when((p == 0) & (i == 0))
    def _():
        macc[...] = jnp.zeros_like(macc)

    @pl.when(p == 0)
    def _():
        g = gu_ref[...]
        macc[...] += lax.dot_general(g, g, (((0,), (0,)), ((), ())),
                                     preferred_element_type=jnp.float32)

    @pl.when((p == 1) & (i == 0))
    def _():
        m = macc[...]
        wl = wl_ref[...]
        wf = wf_ref[...]
        mean_gx = m[0:3, 15:16] / e
        cov_gx = m[0:3, 0:3] / e - mean_gx * mean_gx.T
        mu_l = jnp.dot(wl, mean_gx, preferred_element_type=jnp.float32)
        var_l = jnp.sum(jnp.dot(wl, cov_gx,
                                preferred_element_type=jnp.float32) * wl,
                        axis=1, keepdims=True)
        s_l = gl_ref[...] * lax.rsqrt(var_l + EPS)
        mean_p = m[3:3 + CIN, 15:16] / e
        cov_p = m[3:3 + CIN, 3:3 + CIN] / e - mean_p * mean_p.T
        mu_f = jnp.dot(wf, mean_p, preferred_element_type=jnp.float32)
        var_f = jnp.sum(jnp.dot(wf, cov_p,
                                preferred_element_type=jnp.float32) * wf,
                        axis=1, keepdims=True)
        s_f = gf_ref[...] * lax.rsqrt(var_f + EPS)
        tbuf[:, 0:3] = s_l * wl
        tbuf[:, 3:3 + CIN] = s_f * wf
        tbuf[:, 9:15] = jnp.zeros((C0, 6), jnp.float32)
        tbuf[:, 15:16] = (bl_ref[...] - s_l * mu_l
                          + bf_ref[...] - s_f * mu_f)
        mh[...] = jnp.zeros_like(mh)
        sg[...] = jnp.zeros_like(sg)

    @pl.when(p == 1)
    def _():
        g = gu_ref[...]
        t = tbuf[...]
        h = jnp.maximum(
            lax.dot_general(g, t, (((1,), (1,)), ((), ())),
                            preferred_element_type=jnp.float32), 0.0)
        mh[...] += lax.dot_general(h, h, (((0,), (0,)), ((), ())),
                                   preferred_element_type=jnp.float32)
        sg[...] += lax.dot_general(h, g, (((0,), (0,)), ((), ())),
                                   preferred_element_type=jnp.float32)

    @pl.when((p == 2) & (i == 0))
    def _():
        w1 = w1_ref[...]
        bc1 = bc1_ref[...]
        shv = sg[:, 15:16]
        w1sh = jnp.dot(w1, shv, preferred_element_type=jnp.float32) / e
        mean1 = w1sh + bc1
        ey2 = (jnp.sum(jnp.dot(w1, mh[...],
                               preferred_element_type=jnp.float32) * w1,
                       axis=1, keepdims=True) / e
               + 2.0 * bc1 * w1sh + bc1 * bc1)
        var1 = ey2 - mean1 * mean1
        s1 = g1_ref[...] * lax.rsqrt(var1 + EPS)
        wbuf[:, 0:C0] = s1 * w1
        wbuf[:, C0:C0 + 15] = jnp.zeros((C1, 15), jnp.float32)
        wbuf[:, C0 + 15:C0 + 16] = b1_ref[...] + s1 * (bc1 - mean1)

    @pl.when(p == 2)
    def _():
        g = gu_ref[...]
        t = tbuf[...]
        w = wbuf[...]
        h = jnp.maximum(
            lax.dot_general(g, t, (((1,), (1,)), ((), ())),
                            preferred_element_type=jnp.float32), 0.0)
        hg = jnp.concatenate([h, g], axis=1)
        y = jnp.maximum(
            lax.dot_general(hg, w, (((1,), (1,)), ((), ())),
                            preferred_element_type=jnp.float32), 0.0)
        out_ref[...] = jnp.max(y.reshape(_QF, NS, C1), axis=1)


def _fused(gu, wl, wf, w1, gl, bl, gf, bf, bc1, g1, b1):
    small = lambda r, c: pl.BlockSpec((r, c), lambda p, i: (0, 0))
    return pl.pallas_call(
        _fused_body,
        grid=(3, _NBF),
        in_specs=[
            pl.BlockSpec((_BRF, 16), lambda p, i: (i, 0)),
            small(C0, 3), small(C0, CIN), small(C1, C0),
            small(C0, 1), small(C0, 1), small(C0, 1), small(C0, 1),
            small(C1, 1), small(C1, 1), small(C1, 1),
        ],
        out_specs=pl.BlockSpec((_QF, C1), lambda p, i: (i, 0)),
        out_shape=jax.ShapeDtypeStruct((N, C1), jnp.float32),
        scratch_shapes=[
            pltpu.VMEM((16, 16), jnp.float32),
            pltpu.VMEM((C0, C0), jnp.float32),
            pltpu.VMEM((C0, 16), jnp.float32),
            pltpu.VMEM((C0, 16), jnp.float32),
            pltpu.VMEM((C1, C0 + 16), jnp.float32),
        ],
    )(gu, wl, wf, w1, gl, bl, gf, bf, bc1, g1, b1)


def kernel(xyz, points, offset, W_l0, g_l0, b_l0, W_f0, g_f0, b_f0,
           W1, bc1, g1, b1):
    xyzT = xyz.T
    darr, m128 = _dist(xyz, xyzT)
    gidx_fast, ovf = _sc_topk(darr, m128)
    gidx = lax.cond(jnp.max(ovf) > 128,
                    lambda: _knn_fallback(xyz, xyzT),
                    lambda: gidx_fast)

    pad = jnp.zeros((N, 16 - 3 - CIN), jnp.float32)
    up = jnp.concatenate(
        [xyz, points, pad[:, :-1], jnp.ones((N, 1), jnp.float32)], axis=1)
    vself = jnp.concatenate([xyz, jnp.zeros((N, 13), jnp.float32)], axis=1)
    idx2 = gidx.reshape(NE // 128, 128)

    gu = _sc_gather_sub(up, vself, idx2)

    new_feats = _fused(
        gu, W_l0, W_f0, W1,
        g_l0.reshape(C0, 1), b_l0.reshape(C0, 1),
        g_f0.reshape(C0, 1), b_f0.reshape(C0, 1),
        bc1.reshape(C1, 1), g1.reshape(C1, 1), b1.reshape(C1, 1))
    return (xyz, new_feats, offset)


# speedup vs baseline: 5.1101x; 1.0104x over previous
import functools

import jax
import jax.numpy as jnp
from jax import lax
from jax.experimental import pallas as pl
from jax.experimental.pallas import tpu as pltpu
from jax.experimental.pallas import tpu_sc as plsc

B = 8
NPER = 2048
N = B * NPER
NS = 32
CIN = 6
C0 = 32
C1 = 64
NE = N * NS
EPS = 1e-5

_BQ = 256
_QB = NPER // _BQ


def _knn_body(q_ref, xt_ref, out_ref):
    seg = pl.program_id(0)
    q = q_ref[...]
    xt = xt_ref[...]
    d = (q[:, 0:1] - xt[0:1, :]) ** 2
    d = d + (q[:, 1:2] - xt[1:2, :]) ** 2
    d = d + (q[:, 2:3] - xt[2:3, :]) ** 2
    fiota = lax.broadcasted_iota(jnp.int32, (_BQ, NPER), 1).astype(jnp.float32)
    kiota = lax.broadcasted_iota(jnp.int32, (_BQ, NS), 1)
    inf = jnp.float32(3.4e38)
    fnper = jnp.float32(NPER)

    def step(k, carry):
        dc, acc = carry
        m = jnp.min(dc, axis=1, keepdims=True)
        sel = jnp.min(jnp.where(dc == m, fiota, fnper), axis=1,
                      keepdims=True)
        acc = jnp.where(kiota == k, sel.astype(jnp.int32), acc)
        dc = jnp.where(fiota == sel, inf, dc)
        return dc, acc

    _, acc = lax.fori_loop(0, NS, step,
                           (d, jnp.zeros((_BQ, NS), jnp.int32)))
    out_ref[...] = acc + seg * NPER


def _knn_fallback(xyz, xyzT):
    return pl.pallas_call(
        _knn_body,
        grid=(B, _QB),
        in_specs=[
            pl.BlockSpec((_BQ, 3), lambda s, q: (s * _QB + q, 0)),
            pl.BlockSpec((3, NPER), lambda s, q: (0, s)),
        ],
        out_specs=pl.BlockSpec((_BQ, NS), lambda s, q: (s * _QB + q, 0)),
        out_shape=jax.ShapeDtypeStruct((N, NS), jnp.int32),
    )(xyz, xyzT)


def _dist_body(q_ref, xt_ref, d_ref, m_ref):
    q = q_ref[...]
    xt = xt_ref[...]
    d = (q[:, 0:1] - xt[0:1, :]) ** 2
    d = d + (q[:, 1:2] - xt[1:2, :]) ** 2
    d = d + (q[:, 2:3] - xt[2:3, :]) ** 2
    m = d[:, 0:128]
    for c in range(1, NPER // 128):
        m = jnp.minimum(m, d[:, c * 128:(c + 1) * 128])
    d_ref[...] = d
    m_ref[...] = m


def _dist(xyz, xyzT):
    return pl.pallas_call(
        _dist_body,
        grid=(B, _QB),
        in_specs=[
            pl.BlockSpec((_BQ, 3), lambda s, q: (s * _QB + q, 0)),
            pl.BlockSpec((3, NPER), lambda s, q: (0, s)),
        ],
        out_specs=[
            pl.BlockSpec((_BQ, NPER), lambda s, q: (s * _QB + q, 0)),
            pl.BlockSpec((_BQ, 128), lambda s, q: (s * _QB + q, 0)),
        ],
        out_shape=[
            jax.ShapeDtypeStruct((N, NPER), jnp.float32),
            jax.ShapeDtypeStruct((N, 128), jnp.float32),
        ],
    )(xyz, xyzT)


_INF = 3.4e38
_NW = 32
_RPW1 = N // _NW
_DCH = 8
_MCH = 16


def _mergek16(a, b):
    rb = lax.rev(b, (0,))
    s = jnp.minimum(a, rb)
    t = jnp.maximum(a, rb)
    return lax.sort(s), lax.sort(t)


def _mergek32(a0, a1, b0, b1):
    rb0 = lax.rev(b0, (0,))
    rb1 = lax.rev(b1, (0,))
    s0 = jnp.minimum(a0, rb1)
    s1 = jnp.minimum(a1, rb0)
    u = jnp.minimum(s0, s1)
    v = jnp.maximum(s0, s1)
    return lax.sort(u), lax.sort(v)


def _sc_topk(darr, m128):
    mesh = plsc.VectorSubcoreMesh(core_axis_name="c", subcore_axis_name="s")

    @functools.partial(
        pl.kernel,
        out_type=[
            jax.ShapeDtypeStruct((N, NS), jnp.int32),
            jax.ShapeDtypeStruct((_NW, 16), jnp.int32),
        ],
        mesh=mesh,
        compiler_params=pltpu.CompilerParams(use_tc_tiling_on_sc=False,
                                             needs_layout_passes=False),
        scratch_types=[
            pltpu.VMEM((2, _DCH, NPER), jnp.float32),
            pltpu.VMEM((_RPW1, 128), jnp.float32),
            pltpu.VMEM((NPER,), jnp.int32),
            pltpu.VMEM((128,), jnp.float32),
            pltpu.VMEM((128,), jnp.int32),
            pltpu.VMEM((2, _MCH, NS), jnp.int32),
            pltpu.VMEM((NS,), jnp.int32),
            pltpu.VMEM((16,), jnp.int32),
            pltpu.SemaphoreType.DMA,
            pltpu.SemaphoreType.DMA,
        ],
    )
    def body(d_hbm, m_hbm, gidx_hbm, ovf_hbm, dbuf, mbuf, ramp, cbuf, ibuf,
             ostage, obuf32, obuf, dsem, osem):
        wid = lax.axis_index("s") * 2 + lax.axis_index("c")
        row0 = wid * _RPW1
        segbase = (row0 // NPER) * NPER
        iota16 = lax.iota(jnp.int32, 16)

        def mkramp(j, _):
            ramp[pl.ds(j * 16, 16)] = iota16 + (segbase + j * 16)
            return 0
        lax.fori_loop(0, NPER // 16, mkramp, 0)

        def dcopy(c, buf):
            rb = pl.multiple_of(row0 + c * _DCH, _DCH)
            return pltpu.make_async_copy(
                d_hbm.at[pl.ds(rb, _DCH)], dbuf.at[buf], dsem)

        def ocopy(bi):
            orb = pl.multiple_of(row0 + bi * _MCH, _MCH)
            return pltpu.make_async_copy(
                ostage.at[bi % 2], gidx_hbm.at[pl.ds(orb, _MCH)], osem)

        dcopy(0, 0).start()

        def do_row(dch, rl, rloc, bi):
            rg = bi * _MCH + rloc
            g = [lax.sort(mbuf[rg, pl.ds(gg * 16, 16)]) for gg in range(8)]
            p0 = _mergek16(g[0], g[1])
            p1 = _mergek16(g[2], g[3])
            p2 = _mergek16(g[4], g[5])
            p3 = _mergek16(g[6], g[7])
            q0 = _mergek32(*p0, *p1)
            q1 = _mergek32(*p2, *p3)
            _, f1 = _mergek32(*q0, *q1)
            t2 = jnp.max(f1)

            for gg in range(8):
                cbuf[pl.ds(gg * 16, 16)] = jnp.full((16,), _INF,
                                                    dtype=jnp.float32)

            def comp(jb, offv):
                for jj in range(8):
                    j = jb * 8 + jj
                    v = dbuf[dch, rl, pl.ds(j * 16, 16)]
                    msk = v <= t2
                    cs = jnp.cumsum(msk.astype(jnp.int32))
                    pos = jnp.minimum(offv + cs, 127)
                    plsc.store_scatter(cbuf, [pos], v, mask=msk)
                    plsc.store_scatter(ibuf, [pos],
                                       ramp[pl.ds(j * 16, 16)], mask=msk)
                    offv = offv + plsc.all_reduce_population_count(msk)
                return offv

            offv = lax.fori_loop(0, (NPER // 16) // 8, comp,
                                 jnp.full((16,), -1, jnp.int32))

            g2 = [lax.sort(cbuf[pl.ds(gg * 16, 16)]) for gg in range(8)]
            r0 = _mergek16(g2[0], g2[1])
            r1 = _mergek16(g2[2], g2[3])
            r2 = _mergek16(g2[4], g2[5])
            r3 = _mergek16(g2[6], g2[7])
            w0 = _mergek32(*r0, *r1)
            w1 = _mergek32(*r2, *r3)
            _, s1 = _mergek32(*w0, *w1)
            t32 = jnp.max(s1)

            off2 = jnp.full((16,), -1, jnp.int32)
            for gg in range(8):
                v = cbuf[pl.ds(gg * 16, 16)]
                msk2 = v <= t32
                cs2 = jnp.cumsum(msk2.astype(jnp.int32))
                pos2 = off2 + cs2
                msk3 = msk2 & (pos2 < NS)
                plsc.store_scatter(obuf32, [jnp.minimum(pos2, NS - 1)],
                                   ibuf[pl.ds(gg * 16, 16)], mask=msk3)
                off2 = off2 + plsc.all_reduce_population_count(msk2)
            ostage[bi % 2, rloc, pl.ds(0, 16)] = obuf32[pl.ds(0, 16)]
            ostage[bi % 2, rloc, pl.ds(16, 16)] = obuf32[pl.ds(16, 16)]
            return offv + 1

        mrb0 = pl.multiple_of(row0, _RPW1)
        pltpu.sync_copy(m_hbm.at[pl.ds(mrb0, _RPW1)], mbuf)

        def blk16(bi, ofmax):
            @pl.when(bi >= 2)
            def _():
                ocopy(0).wait()
            for rb4 in range(_MCH // _DCH):
                c = bi * (_MCH // _DCH) + rb4
                dcopy(c, c % 2).wait()

                @pl.when(c + 1 < _RPW1 // _DCH)
                def _():
                    dcopy(c + 1, (c + 1) % 2).start()
                for rl in range(_DCH):
                    cnt = do_row(c % 2, rl, rb4 * _DCH + rl, bi)
                    ofmax = jnp.maximum(ofmax, cnt)
            ocopy(bi).start()
            return ofmax

        ofmax = lax.fori_loop(0, _RPW1 // _MCH, blk16,
                              jnp.zeros((16,), jnp.int32))
        ocopy(0).wait()
        ocopy(0).wait()
        obuf[pl.ds(0, 16)] = ofmax
        pltpu.sync_copy(obuf, ovf_hbm.at[wid])

    return body(darr, m128)


_RPW = NE // _NW
_CH_ROWS = 2048
_CH_Q = _CH_ROWS // NS
_NCH = _RPW // _CH_ROWS
_GPC = _CH_ROWS // 128


def _sc_gather_sub(up, vself, idx2):
    mesh = plsc.VectorSubcoreMesh(core_axis_name="c", subcore_axis_name="s")

    @functools.partial(
        pl.kernel,
        out_type=jax.ShapeDtypeStruct((NE, 16), jnp.float32),
        mesh=mesh,
        compiler_params=pltpu.CompilerParams(use_tc_tiling_on_sc=False),
        scratch_types=[
            pltpu.VMEM((_GPC, 128), jnp.int32),
            pltpu.VMEM((_CH_ROWS, 16), jnp.float32),
            pltpu.VMEM((_CH_Q, 16), jnp.float32),
            pltpu.SemaphoreType.DMA,
        ],
    )
    def body(up_hbm, vs_hbm, idx_hbm, out_hbm, idx_v, rows_v, vself_v, sem):
        wid = lax.axis_index("s") * 2 + lax.axis_index("c")
        row_base = wid * _RPW
        q_base = wid * (_RPW // NS)

        def chunk(c, _):
            rb = pl.multiple_of(row_base + c * _CH_ROWS, _CH_ROWS)
            qb = pl.multiple_of(q_base + c * _CH_Q, _CH_Q)
            ib = pl.multiple_of(rb // 128, _GPC)
            pltpu.sync_copy(idx_hbm.at[pl.ds(ib, _GPC)], idx_v)
            cps = [
                pltpu.make_async_copy(
                    up_hbm.at[idx_v.at[j]],
                    rows_v.at[pl.ds(j * 128, 128)],
                    sem,
                )
                for j in range(_GPC)
            ]
            for cp in cps:
                cp.start()
            for cp in cps:
                cp.wait()
            pltpu.sync_copy(vs_hbm.at[pl.ds(qb, _CH_Q)], vself_v)

            def subq(i, _):
                v = vself_v[i]
                base = i * NS
                for s2 in range(NS):
                    rows_v[base + s2] = rows_v[base + s2] - v
                return 0

            lax.fori_loop(0, _CH_Q, subq, 0)
            pltpu.sync_copy(rows_v, out_hbm.at[pl.ds(rb, _CH_ROWS)])
            return 0

        lax.fori_loop(0, _NCH, chunk, 0)

    return body(up, vself, idx2)


_BRF = 8192
_NBF = NE // _BRF
_QF = _BRF // NS


def _fused_body(gu_ref, wl_ref, wf_ref, w1_ref, gl_ref, bl_ref, gf_ref,
                bf_ref, bc1_ref, g1_ref, b1_ref, out_ref,
                macc, mh, sg, tbuf, wbuf):
    p = pl.program_id(0)
    i = pl.program_id(1)
    e = jnp.float32(NE)

    @pl.when((p == 0) & (i == 0))
    def _():
        macc[...] = jnp.zeros_like(macc)

    @pl.when(p == 0)
    def _():
        g = gu_ref[...]
        macc[...] += lax.dot_general(g, g, (((0,), (0,)), ((), ())),
                                     preferred_element_type=jnp.float32)

    @pl.when((p == 1) & (i == 0))
    def _():
        m = macc[...]
        wl = wl_ref[...]
        wf = wf_ref[...]
        mean_gx = m[0:3, 15:16] / e
        cov_gx = m[0:3, 0:3] / e - mean_gx * mean_gx.T
        mu_l = jnp.dot(wl, mean_gx, preferred_element_type=jnp.float32)
        var_l = jnp.sum(jnp.dot(wl, cov_gx,
                                preferred_element_type=jnp.float32) * wl,
                        axis=1, keepdims=True)
        s_l = gl_ref[...] * lax.rsqrt(var_l + EPS)
        mean_p = m[3:3 + CIN, 15:16] / e
        cov_p = m[3:3 + CIN, 3:3 + CIN] / e - mean_p * mean_p.T
        mu_f = jnp.dot(wf, mean_p, preferred_element_type=jnp.float32)
        var_f = jnp.sum(jnp.dot(wf, cov_p,
                                preferred_element_type=jnp.float32) * wf,
                        axis=1, keepdims=True)
        s_f = gf_ref[...] * lax.rsqrt(var_f + EPS)
        tbuf[:, 0:3] = s_l * wl
        tbuf[:, 3:3 + CIN] = s_f * wf
        tbuf[:, 9:15] = jnp.zeros((C0, 6), jnp.float32)
        tbuf[:, 15:16] = (bl_ref[...] - s_l * mu_l
                          + bf_ref[...] - s_f * mu_f)
        mh[...] = jnp.zeros_like(mh)
        sg[...] = jnp.zeros_like(sg)

    @pl.when(p == 1)
    def _():
        g = gu_ref[...]
        t = tbuf[...]
        h = jnp.maximum(
            lax.dot_general(g, t, (((1,), (1,)), ((), ())),
                            preferred_element_type=jnp.float32), 0.0)
        mh[...] += lax.dot_general(h, h, (((0,), (0,)), ((), ())),
                                   preferred_element_type=jnp.float32)
        sg[...] += lax.dot_general(h, g, (((0,), (0,)), ((), ())),
                                   preferred_element_type=jnp.float32)

    @pl.when((p == 2) & (i == 0))
    def _():
        w1 = w1_ref[...]
        bc1 = bc1_ref[...]
        shv = sg[:, 15:16]
        w1sh = jnp.dot(w1, shv, preferred_element_type=jnp.float32) / e
        mean1 = w1sh + bc1
        ey2 = (jnp.sum(jnp.dot(w1, mh[...],
                               preferred_element_type=jnp.float32) * w1,
                       axis=1, keepdims=True) / e
               + 2.0 * bc1 * w1sh + bc1 * bc1)
        var1 = ey2 - mean1 * mean1
        s1 = g1_ref[...] * lax.rsqrt(var1 + EPS)
        wbuf[:, 0:C0] = s1 * w1
        wbuf[:, C0:C0 + 15] = jnp.zeros((C1, 15), jnp.float32)
        wbuf[:, C0 + 15:C0 + 16] = b1_ref[...] + s1 * (bc1 - mean1)

    @pl.when(p == 2)
    def _():
        g = gu_ref[...]
        t = tbuf[...]
        w = wbuf[...]
        h = jnp.maximum(
            lax.dot_general(g, t, (((1,), (1,)), ((), ())),
                            preferred_element_type=jnp.float32), 0.0)
        hg = jnp.concatenate([h, g], axis=1)
        y = jnp.maximum(
            lax.dot_general(hg, w, (((1,), (1,)), ((), ())),
                            preferred_element_type=jnp.float32), 0.0)
        out_ref[...] = jnp.max(y.reshape(_QF, NS, C1), axis=1)


def _fused(gu, wl, wf, w1, gl, bl, gf, bf, bc1, g1, b1):
    small = lambda r, c: pl.BlockSpec((r, c), lambda p, i: (0, 0))
    return pl.pallas_call(
        _fused_body,
        grid=(3, _NBF),
        in_specs=[
            pl.BlockSpec((_BRF, 16), lambda p, i: (i, 0)),
            small(C0, 3), small(C0, CIN), small(C1, C0),
            small(C0, 1), small(C0, 1), small(C0, 1), small(C0, 1),
            small(C1, 1), small(C1, 1), small(C1, 1),
        ],
        out_specs=pl.BlockSpec((_QF, C1), lambda p, i: (i, 0)),
        out_shape=jax.ShapeDtypeStruct((N, C1), jnp.float32),
        scratch_shapes=[
            pltpu.VMEM((16, 16), jnp.float32),
            pltpu.VMEM((C0, C0), jnp.float32),
            pltpu.VMEM((C0, 16), jnp.float32),
            pltpu.VMEM((C0, 16), jnp.float32),
            pltpu.VMEM((C1, C0 + 16), jnp.float32),
        ],
    )(gu, wl, wf, w1, gl, bl, gf, bf, bc1, g1, b1)


def kernel(xyz, points, offset, W_l0, g_l0, b_l0, W_f0, g_f0, b_f0,
           W1, bc1, g1, b1):
    xyzT = xyz.T
    darr, m128 = _dist(xyz, xyzT)
    gidx_fast, ovf = _sc_topk(darr, m128)
    gidx = lax.cond(jnp.max(ovf) > 128,
                    lambda: _knn_fallback(xyz, xyzT),
                    lambda: gidx_fast)

    pad = jnp.zeros((N, 16 - 3 - CIN), jnp.float32)
    up = jnp.concatenate(
        [xyz, points, pad[:, :-1], jnp.ones((N, 1), jnp.float32)], axis=1)
    vself = jnp.concatenate([xyz, jnp.zeros((N, 13), jnp.float32)], axis=1)
    idx2 = gidx.reshape(NE // 128, 128)

    gu = _sc_gather_sub(up, vself, idx2)

    new_feats = _fused(
        gu, W_l0, W_f0, W1,
        g_l0.reshape(C0, 1), b_l0.reshape(C0, 1),
        g_f0.reshape(C0, 1), b_f0.reshape(C0, 1),
        bc1.reshape(C1, 1), g1.reshape(C1, 1), b1.reshape(C1, 1))
    return (xyz, new_feats, offset)


# speedup vs baseline: 5.3304x; 1.0431x over previous
import functools

import jax
import jax.numpy as jnp
from jax import lax
from jax.experimental import pallas as pl
from jax.experimental.pallas import tpu as pltpu
from jax.experimental.pallas import tpu_sc as plsc

B = 8
NPER = 2048
N = B * NPER
NS = 32
CIN = 6
C0 = 32
C1 = 64
NE = N * NS
EPS = 1e-5

_BQ = 256
_QB = NPER // _BQ


def _knn_body(q_ref, xt_ref, out_ref):
    seg = pl.program_id(0)
    q = q_ref[...]
    xt = xt_ref[...]
    d = (q[:, 0:1] - xt[0:1, :]) ** 2
    d = d + (q[:, 1:2] - xt[1:2, :]) ** 2
    d = d + (q[:, 2:3] - xt[2:3, :]) ** 2
    fiota = lax.broadcasted_iota(jnp.int32, (_BQ, NPER), 1).astype(jnp.float32)
    kiota = lax.broadcasted_iota(jnp.int32, (_BQ, NS), 1)
    inf = jnp.float32(3.4e38)
    fnper = jnp.float32(NPER)

    def step(k, carry):
        dc, acc = carry
        m = jnp.min(dc, axis=1, keepdims=True)
        sel = jnp.min(jnp.where(dc == m, fiota, fnper), axis=1,
                      keepdims=True)
        acc = jnp.where(kiota == k, sel.astype(jnp.int32), acc)
        dc = jnp.where(fiota == sel, inf, dc)
        return dc, acc

    _, acc = lax.fori_loop(0, NS, step,
                           (d, jnp.zeros((_BQ, NS), jnp.int32)))
    out_ref[...] = acc + seg * NPER


def _knn_fallback(xyz, xyzT):
    return pl.pallas_call(
        _knn_body,
        grid=(B, _QB),
        in_specs=[
            pl.BlockSpec((_BQ, 3), lambda s, q: (s * _QB + q, 0)),
            pl.BlockSpec((3, NPER), lambda s, q: (0, s)),
        ],
        out_specs=pl.BlockSpec((_BQ, NS), lambda s, q: (s * _QB + q, 0)),
        out_shape=jax.ShapeDtypeStruct((N, NS), jnp.int32),
    )(xyz, xyzT)


def _dist_body(q_ref, xt_ref, d_ref, t2_ref, p_ref, cnt_ref):
    q = q_ref[...]
    xt = xt_ref[...]
    d = (q[:, 0:1] - xt[0:1, :]) ** 2
    d = d + (q[:, 1:2] - xt[1:2, :]) ** 2
    d = d + (q[:, 2:3] - xt[2:3, :]) ** 2
    m = d[:, 0:128]
    for c in range(1, NPER // 128):
        m = jnp.minimum(m, d[:, c * 128:(c + 1) * 128])
    mt = m.T
    inf = jnp.float32(3.4e38)

    def tstep(k, carry):
        cur, _ = carry
        mn = jnp.min(cur, axis=0, keepdims=True)
        cur = jnp.where(cur == mn, inf, cur)
        return cur, mn

    _, t2row = lax.fori_loop(0, NS, tstep, (mt, mt[0:1, :]))
    t2 = t2row.T
    mask = d <= t2
    mi = mask.astype(jnp.int32)
    lanemod = lax.broadcasted_iota(jnp.int32, (_BQ, NPER), 1) % 16
    pre = mi
    for s in (1, 2, 4, 8):
        shifted = jnp.concatenate(
            [jnp.zeros((_BQ, s), jnp.int32), pre[:, :NPER - s]], axis=1)
        pre = pre + jnp.where(lanemod >= s, shifted, 0)
    p_ref[...] = pre - 1
    d_ref[...] = d
    t2_ref[...] = jnp.broadcast_to(t2, (_BQ, 16))
    cnt_ref[...] = jnp.broadcast_to(
        jnp.sum(mi, axis=1, keepdims=True), (_BQ, 8))


def _dist(xyz, xyzT):
    return pl.pallas_call(
        _dist_body,
        grid=(B, _QB),
        in_specs=[
            pl.BlockSpec((_BQ, 3), lambda s, q: (s * _QB + q, 0)),
            pl.BlockSpec((3, NPER), lambda s, q: (0, s)),
        ],
        out_specs=[
            pl.BlockSpec((_BQ, NPER), lambda s, q: (s * _QB + q, 0)),
            pl.BlockSpec((_BQ, 16), lambda s, q: (s * _QB + q, 0)),
            pl.BlockSpec((_BQ, NPER), lambda s, q: (s * _QB + q, 0)),
            pl.BlockSpec((_BQ, 8), lambda s, q: (s * _QB + q, 0)),
        ],
        out_shape=[
            jax.ShapeDtypeStruct((N, NPER), jnp.float32),
            jax.ShapeDtypeStruct((N, 16), jnp.float32),
            jax.ShapeDtypeStruct((N, NPER), jnp.int32),
            jax.ShapeDtypeStruct((N, 8), jnp.int32),
        ],
    )(xyz, xyzT)


_INF = 3.4e38
_NW = 32
_RPW1 = N // _NW
_DCH = 8
_MCH = 16


def _mergek16(a, b):
    rb = lax.rev(b, (0,))
    s = jnp.minimum(a, rb)
    t = jnp.maximum(a, rb)
    return lax.sort(s), lax.sort(t)


def _mergek32(a0, a1, b0, b1):
    rb0 = lax.rev(b0, (0,))
    rb1 = lax.rev(b1, (0,))
    s0 = jnp.minimum(a0, rb1)
    s1 = jnp.minimum(a1, rb0)
    u = jnp.minimum(s0, s1)
    v = jnp.maximum(s0, s1)
    return lax.sort(u), lax.sort(v)


def _sc_topk(darr, t2arr, posarr):
    mesh = plsc.VectorSubcoreMesh(core_axis_name="c", subcore_axis_name="s")

    @functools.partial(
        pl.kernel,
        out_type=jax.ShapeDtypeStruct((N, NS), jnp.int32),
        mesh=mesh,
        compiler_params=pltpu.CompilerParams(use_tc_tiling_on_sc=False,
                                             needs_layout_passes=False),
        scratch_types=[
            pltpu.VMEM((2, _DCH, NPER), jnp.float32),
            pltpu.VMEM((2, _DCH, NPER), jnp.int32),
            pltpu.VMEM((_RPW1, 16), jnp.float32),
            pltpu.VMEM((NPER,), jnp.int32),
            pltpu.VMEM((128,), jnp.float32),
            pltpu.VMEM((128,), jnp.int32),
            pltpu.VMEM((2, _MCH, NS), jnp.int32),
            pltpu.VMEM((NS,), jnp.int32),
            pltpu.SemaphoreType.DMA,
            pltpu.SemaphoreType.DMA,
        ],
    )
    def body(d_hbm, t2_hbm, pos_hbm, gidx_hbm, dbuf, pbuf, t2buf, ramp,
             cbuf, ibuf, ostage, obuf32, dsem, osem):
        wid = lax.axis_index("s") * 2 + lax.axis_index("c")
        row0 = wid * _RPW1
        segbase = (row0 // NPER) * NPER
        iota16 = lax.iota(jnp.int32, 16)

        def mkramp(j, _):
            ramp[pl.ds(j * 16, 16)] = iota16 + (segbase + j * 16)
            return 0
        lax.fori_loop(0, NPER // 16, mkramp, 0)

        def dcopy(c, buf):
            rb = pl.multiple_of(row0 + c * _DCH, _DCH)
            return (pltpu.make_async_copy(
                        d_hbm.at[pl.ds(rb, _DCH)], dbuf.at[buf], dsem),
                    pltpu.make_async_copy(
                        pos_hbm.at[pl.ds(rb, _DCH)], pbuf.at[buf], dsem))

        def ocopy(bi):
            orb = pl.multiple_of(row0 + bi * _MCH, _MCH)
            return pltpu.make_async_copy(
                ostage.at[bi % 2], gidx_hbm.at[pl.ds(orb, _MCH)], osem)

        for cp in dcopy(0, 0):
            cp.start()

        def do_row(dch, rl, rloc, bi):
            rg = bi * _MCH + rloc
            t2v = t2buf[rg]

            for gg in range(8):
                cbuf[pl.ds(gg * 16, 16)] = jnp.full((16,), _INF,
                                                    dtype=jnp.float32)

            def comp(jb, offv):
                for jj in range(8):
                    j = jb * 8 + jj
                    v = dbuf[dch, rl, pl.ds(j * 16, 16)]
                    pv = pbuf[dch, rl, pl.ds(j * 16, 16)]
                    msk = v <= t2v
                    pos = jnp.minimum(offv + pv, 127)
                    plsc.store_scatter(cbuf, [pos], v, mask=msk)
                    plsc.store_scatter(ibuf, [pos],
                                       ramp[pl.ds(j * 16, 16)], mask=msk)
                    offv = offv + plsc.all_reduce_population_count(msk)
                return offv

            offv = lax.fori_loop(0, (NPER // 16) // 8, comp,
                                 jnp.zeros((16,), jnp.int32))

            g2 = [lax.sort(cbuf[pl.ds(gg * 16, 16)]) for gg in range(8)]
            r0 = _mergek16(g2[0], g2[1])
            r1 = _mergek16(g2[2], g2[3])
            r2 = _mergek16(g2[4], g2[5])
            r3 = _mergek16(g2[6], g2[7])
            w0 = _mergek32(*r0, *r1)
            w1 = _mergek32(*r2, *r3)
            _, s1 = _mergek32(*w0, *w1)
            t32 = jnp.max(s1)

            off2 = jnp.full((16,), -1, jnp.int32)
            for gg in range(8):
                v = cbuf[pl.ds(gg * 16, 16)]
                msk2 = v <= t32
                cs2 = jnp.cumsum(msk2.astype(jnp.int32))
                pos2 = off2 + cs2
                msk3 = msk2 & (pos2 < NS)
                plsc.store_scatter(obuf32, [jnp.minimum(pos2, NS - 1)],
                                   ibuf[pl.ds(gg * 16, 16)], mask=msk3)
                off2 = off2 + plsc.all_reduce_population_count(msk2)
            ostage[bi % 2, rloc, pl.ds(0, 16)] = obuf32[pl.ds(0, 16)]
            ostage[bi % 2, rloc, pl.ds(16, 16)] = obuf32[pl.ds(16, 16)]
            return 0

        trb0 = pl.multiple_of(row0, _RPW1)
        pltpu.sync_copy(t2_hbm.at[pl.ds(trb0, _RPW1)], t2buf)

        def blk16(bi, _):
            @pl.when(bi >= 2)
            def _():
                ocopy(0).wait()
            for rb4 in range(_MCH // _DCH):
                c = bi * (_MCH // _DCH) + rb4
                for cp in dcopy(c, c % 2):
                    cp.wait()

                @pl.when(c + 1 < _RPW1 // _DCH)
                def _():
                    for cp in dcopy(c + 1, (c + 1) % 2):
                        cp.start()
                for rl in range(_DCH):
                    do_row(c % 2, rl, rb4 * _DCH + rl, bi)
            ocopy(bi).start()
            return 0

        lax.fori_loop(0, _RPW1 // _MCH, blk16, 0)
        ocopy(0).wait()
        ocopy(0).wait()

    return body(darr, t2arr, posarr)


_RPW = NE // _NW
_CH_ROWS = 2048
_CH_Q = _CH_ROWS // NS
_NCH = _RPW // _CH_ROWS
_GPC = _CH_ROWS // 128


def _sc_gather_sub(up, vself, idx2):
    mesh = plsc.VectorSubcoreMesh(core_axis_name="c", subcore_axis_name="s")

    @functools.partial(
        pl.kernel,
        out_type=jax.ShapeDtypeStruct((NE, 16), jnp.float32),
        mesh=mesh,
        compiler_params=pltpu.CompilerParams(use_tc_tiling_on_sc=False),
        scratch_types=[
            pltpu.VMEM((_GPC, 128), jnp.int32),
            pltpu.VMEM((_CH_ROWS, 16), jnp.float32),
            pltpu.VMEM((_CH_Q, 16), jnp.float32),
            pltpu.SemaphoreType.DMA,
        ],
    )
    def body(up_hbm, vs_hbm, idx_hbm, out_hbm, idx_v, rows_v, vself_v, sem):
        wid = lax.axis_index("s") * 2 + lax.axis_index("c")
        row_base = wid * _RPW
        q_base = wid * (_RPW // NS)

        def chunk(c, _):
            rb = pl.multiple_of(row_base + c * _CH_ROWS, _CH_ROWS)
            qb = pl.multiple_of(q_base + c * _CH_Q, _CH_Q)
            ib = pl.multiple_of(rb // 128, _GPC)
            pltpu.sync_copy(idx_hbm.at[pl.ds(ib, _GPC)], idx_v)
            cps = [
                pltpu.make_async_copy(
                    up_hbm.at[idx_v.at[j]],
                    rows_v.at[pl.ds(j * 128, 128)],
                    sem,
                )
                for j in range(_GPC)
            ]
            for cp in cps:
                cp.start()
            for cp in cps:
                cp.wait()
            pltpu.sync_copy(vs_hbm.at[pl.ds(qb, _CH_Q)], vself_v)

            def subq(i, _):
                v = vself_v[i]
                base = i * NS
                for s2 in range(NS):
                    rows_v[base + s2] = rows_v[base + s2] - v
                return 0

            lax.fori_loop(0, _CH_Q, subq, 0)
            pltpu.sync_copy(rows_v, out_hbm.at[pl.ds(rb, _CH_ROWS)])
            return 0

        lax.fori_loop(0, _NCH, chunk, 0)

    return body(up, vself, idx2)


_BRF = 8192
_NBF = NE // _BRF
_QF = _BRF // NS


def _fused_body(gu_ref, wl_ref, wf_ref, w1_ref, gl_ref, bl_ref, gf_ref,
                bf_ref, bc1_ref, g1_ref, b1_ref, out_ref,
                macc, mh, sg, tbuf, wbuf):
    p = pl.program_id(0)
    i = pl.program_id(1)
    e = jnp.float32(NE)

    @pl.when((p == 0) & (i == 0))
    def _():
        macc[...] = jnp.zeros_like(macc)

    @pl.when(p == 0)
    def _():
        g = gu_ref[...]
        macc[...] += lax.dot_general(g, g, (((0,), (0,)), ((), ())),
                                     preferred_element_type=jnp.float32)

    @pl.when((p == 1) & (i == 0))
    def _():
        m = macc[...]
        wl = wl_ref[...]
        wf = wf_ref[...]
        mean_gx = m[0:3, 15:16] / e
        cov_gx = m[0:3, 0:3] / e - mean_gx * mean_gx.T
        mu_l = jnp.dot(wl, mean_gx, preferred_element_type=jnp.float32)
        var_l = jnp.sum(jnp.dot(wl, cov_gx,
                                preferred_element_type=jnp.float32) * wl,
                        axis=1, keepdims=True)
        s_l = gl_ref[...] * lax.rsqrt(var_l + EPS)
        mean_p = m[3:3 + CIN, 15:16] / e
        cov_p = m[3:3 + CIN, 3:3 + CIN] / e - mean_p * mean_p.T
        mu_f = jnp.dot(wf, mean_p, preferred_element_type=jnp.float32)
        var_f = jnp.sum(jnp.dot(wf, cov_p,
                                preferred_element_type=jnp.float32) * wf,
                        axis=1, keepdims=True)
        s_f = gf_ref[...] * lax.rsqrt(var_f + EPS)
        tbuf[:, 0:3] = s_l * wl
        tbuf[:, 3:3 + CIN] = s_f * wf
        tbuf[:, 9:15] = jnp.zeros((C0, 6), jnp.float32)
        tbuf[:, 15:16] = (bl_ref[...] - s_l * mu_l
                          + bf_ref[...] - s_f * mu_f)
        mh[...] = jnp.zeros_like(mh)
        sg[...] = jnp.zeros_like(sg)

    @pl.when(p == 1)
    def _():
        g = gu_ref[...]
        t = tbuf[...]
        h = jnp.maximum(
            lax.dot_general(g, t, (((1,), (1,)), ((), ())),
                            preferred_element_type=jnp.float32), 0.0)
        mh[...] += lax.dot_general(h, h, (((0,), (0,)), ((), ())),
                                   preferred_element_type=jnp.float32)
        sg[...] += lax.dot_general(h, g, (((0,), (0,)), ((), ())),
                                   preferred_element_type=jnp.float32)

    @pl.when((p == 2) & (i == 0))
    def _():
        w1 = w1_ref[...]
        bc1 = bc1_ref[...]
        shv = sg[:, 15:16]
        w1sh = jnp.dot(w1, shv, preferred_element_type=jnp.float32) / e
        mean1 = w1sh + bc1
        ey2 = (jnp.sum(jnp.dot(w1, mh[...],
                               preferred_element_type=jnp.float32) * w1,
                       axis=1, keepdims=True) / e
               + 2.0 * bc1 * w1sh + bc1 * bc1)
        var1 = ey2 - mean1 * mean1
        s1 = g1_ref[...] * lax.rsqrt(var1 + EPS)
        wbuf[:, 0:C0] = s1 * w1
        wbuf[:, C0:C0 + 15] = jnp.zeros((C1, 15), jnp.float32)
        wbuf[:, C0 + 15:C0 + 16] = b1_ref[...] + s1 * (bc1 - mean1)

    @pl.when(p == 2)
    def _():
        g = gu_ref[...]
        t = tbuf[...]
        w = wbuf[...]
        h = jnp.maximum(
            lax.dot_general(g, t, (((1,), (1,)), ((), ())),
                            preferred_element_type=jnp.float32), 0.0)
        hg = jnp.concatenate([h, g], axis=1)
        y = jnp.maximum(
            lax.dot_general(hg, w, (((1,), (1,)), ((), ())),
                            preferred_element_type=jnp.float32), 0.0)
        out_ref[...] = jnp.max(y.reshape(_QF, NS, C1), axis=1)


def _fused(gu, wl, wf, w1, gl, bl, gf, bf, bc1, g1, b1):
    small = lambda r, c: pl.BlockSpec((r, c), lambda p, i: (0, 0))
    return pl.pallas_call(
        _fused_body,
        grid=(3, _NBF),
        in_specs=[
            pl.BlockSpec((_BRF, 16), lambda p, i: (i, 0)),
            small(C0, 3), small(C0, CIN), small(C1, C0),
            small(C0, 1), small(C0, 1), small(C0, 1), small(C0, 1),
            small(C1, 1), small(C1, 1), small(C1, 1),
        ],
        out_specs=pl.BlockSpec((_QF, C1), lambda p, i: (i, 0)),
        out_shape=jax.ShapeDtypeStruct((N, C1), jnp.float32),
        scratch_shapes=[
            pltpu.VMEM((16, 16), jnp.float32),
            pltpu.VMEM((C0, C0), jnp.float32),
            pltpu.VMEM((C0, 16), jnp.float32),
            pltpu.VMEM((C0, 16), jnp.float32),
            pltpu.VMEM((C1, C0 + 16), jnp.float32),
        ],
    )(gu, wl, wf, w1, gl, bl, gf, bf, bc1, g1, b1)


def kernel(xyz, points, offset, W_l0, g_l0, b_l0, W_f0, g_f0, b_f0,
           W1, bc1, g1, b1):
    xyzT = xyz.T
    darr, t2arr, posarr, cnt = _dist(xyz, xyzT)
    gidx_fast = _sc_topk(darr, t2arr, posarr)
    gidx = lax.cond(jnp.max(cnt) > 128,
                    lambda: _knn_fallback(xyz, xyzT),
                    lambda: gidx_fast)

    pad = jnp.zeros((N, 16 - 3 - CIN), jnp.float32)
    up = jnp.concatenate(
        [xyz, points, pad[:, :-1], jnp.ones((N, 1), jnp.float32)], axis=1)
    vself = jnp.concatenate([xyz, jnp.zeros((N, 13), jnp.float32)], axis=1)
    idx2 = gidx.reshape(NE // 128, 128)

    gu = _sc_gather_sub(up, vself, idx2)

    new_feats = _fused(
        gu, W_l0, W_f0, W1,
        g_l0.reshape(C0, 1), b_l0.reshape(C0, 1),
        g_f0.reshape(C0, 1), b_f0.reshape(C0, 1),
        bc1.reshape(C1, 1), g1.reshape(C1, 1), b1.reshape(C1, 1))
    return (xyz, new_feats, offset)


# speedup vs baseline: 5.4720x; 1.0266x over previous
import functools

import jax
import jax.numpy as jnp
from jax import lax
from jax.experimental import pallas as pl
from jax.experimental.pallas import tpu as pltpu
from jax.experimental.pallas import tpu_sc as plsc

B = 8
NPER = 2048
N = B * NPER
NS = 32
CIN = 6
C0 = 32
C1 = 64
NE = N * NS
EPS = 1e-5

_BQ = 256
_QB = NPER // _BQ


def _knn_body(q_ref, xt_ref, out_ref):
    seg = pl.program_id(0)
    q = q_ref[...]
    xt = xt_ref[...]
    d = (q[:, 0:1] - xt[0:1, :]) ** 2
    d = d + (q[:, 1:2] - xt[1:2, :]) ** 2
    d = d + (q[:, 2:3] - xt[2:3, :]) ** 2
    fiota = lax.broadcasted_iota(jnp.int32, (_BQ, NPER), 1).astype(jnp.float32)
    kiota = lax.broadcasted_iota(jnp.int32, (_BQ, NS), 1)
    inf = jnp.float32(3.4e38)
    fnper = jnp.float32(NPER)

    def step(k, carry):
        dc, acc = carry
        m = jnp.min(dc, axis=1, keepdims=True)
        sel = jnp.min(jnp.where(dc == m, fiota, fnper), axis=1,
                      keepdims=True)
        acc = jnp.where(kiota == k, sel.astype(jnp.int32), acc)
        dc = jnp.where(fiota == sel, inf, dc)
        return dc, acc

    _, acc = lax.fori_loop(0, NS, step,
                           (d, jnp.zeros((_BQ, NS), jnp.int32)))
    out_ref[...] = acc + seg * NPER


def _knn_fallback(xyz, xyzT):
    return pl.pallas_call(
        _knn_body,
        grid=(B, _QB),
        in_specs=[
            pl.BlockSpec((_BQ, 3), lambda s, q: (s * _QB + q, 0)),
            pl.BlockSpec((3, NPER), lambda s, q: (0, s)),
        ],
        out_specs=pl.BlockSpec((_BQ, NS), lambda s, q: (s * _QB + q, 0)),
        out_shape=jax.ShapeDtypeStruct((N, NS), jnp.int32),
    )(xyz, xyzT)


def _dist_body(q_ref, xt_ref, d_ref, cnt_ref):
    q = q_ref[...]
    xt = xt_ref[...]
    d = (q[:, 0:1] - xt[0:1, :]) ** 2
    d = d + (q[:, 1:2] - xt[1:2, :]) ** 2
    d = d + (q[:, 2:3] - xt[2:3, :]) ** 2
    m = d[:, 0:128]
    for c in range(1, NPER // 128):
        m = jnp.minimum(m, d[:, c * 128:(c + 1) * 128])
    mt = m.T
    inf = jnp.float32(3.4e38)

    def tstep(k, carry):
        cur, _ = carry
        mn = jnp.min(cur, axis=0, keepdims=True)
        cur = jnp.where(cur == mn, inf, cur)
        return cur, mn

    _, t2row = lax.fori_loop(0, NS, tstep, (mt, mt[0:1, :]))
    t2 = t2row.T
    mask = d <= t2
    mi = mask.astype(jnp.int32)
    lanemod = lax.broadcasted_iota(jnp.int32, (_BQ, NPER), 1) % 16
    pre = mi
    for s in (1, 2, 4, 8):
        shifted = jnp.concatenate(
            [jnp.zeros((_BQ, s), jnp.int32), pre[:, :NPER - s]], axis=1)
        pre = pre + jnp.where(lanemod >= s, shifted, 0)
    du = lax.bitcast_convert_type(d, jnp.int32)
    dpb = (du & jnp.int32(~15)) | (pre - 1)
    d_ref[...] = jnp.where(mask, lax.bitcast_convert_type(dpb, jnp.float32),
                           inf)
    cnt_ref[...] = jnp.broadcast_to(
        jnp.sum(mi, axis=1, keepdims=True), (_BQ, 8))


def _dist(xyz, xyzT):
    return pl.pallas_call(
        _dist_body,
        grid=(B, _QB),
        in_specs=[
            pl.BlockSpec((_BQ, 3), lambda s, q: (s * _QB + q, 0)),
            pl.BlockSpec((3, NPER), lambda s, q: (0, s)),
        ],
        out_specs=[
            pl.BlockSpec((_BQ, NPER), lambda s, q: (s * _QB + q, 0)),
            pl.BlockSpec((_BQ, 8), lambda s, q: (s * _QB + q, 0)),
        ],
        out_shape=[
            jax.ShapeDtypeStruct((N, NPER), jnp.float32),
            jax.ShapeDtypeStruct((N, 8), jnp.int32),
        ],
    )(xyz, xyzT)


_INF = 3.4e38
_NW = 32
_RPW1 = N // _NW
_DCH = 16
_MCH = 16


def _mergek16(a, b):
    rb = lax.rev(b, (0,))
    s = jnp.minimum(a, rb)
    t = jnp.maximum(a, rb)
    return lax.sort(s), lax.sort(t)


def _mergek32(a0, a1, b0, b1):
    rb0 = lax.rev(b0, (0,))
    rb1 = lax.rev(b1, (0,))
    s0 = jnp.minimum(a0, rb1)
    s1 = jnp.minimum(a1, rb0)
    u = jnp.minimum(s0, s1)
    v = jnp.maximum(s0, s1)
    return lax.sort(u), lax.sort(v)


def _sc_topk(darr):
    mesh = plsc.VectorSubcoreMesh(core_axis_name="c", subcore_axis_name="s")

    @functools.partial(
        pl.kernel,
        out_type=jax.ShapeDtypeStruct((N, NS), jnp.int32),
        mesh=mesh,
        compiler_params=pltpu.CompilerParams(use_tc_tiling_on_sc=False,
                                             needs_layout_passes=False),
        scratch_types=[
            pltpu.VMEM((2, _DCH, NPER), jnp.float32),
            pltpu.VMEM((NPER,), jnp.int32),
            pltpu.VMEM((128,), jnp.float32),
            pltpu.VMEM((128,), jnp.int32),
            pltpu.VMEM((2, _MCH, NS), jnp.int32),
            pltpu.VMEM((NS,), jnp.int32),
            pltpu.SemaphoreType.DMA,
            pltpu.SemaphoreType.DMA,
        ],
    )
    def body(d_hbm, gidx_hbm, dbuf, ramp,
             cbuf, ibuf, ostage, obuf32, dsem, osem):
        wid = lax.axis_index("s") * 2 + lax.axis_index("c")
        row0 = wid * _RPW1
        segbase = (row0 // NPER) * NPER
        iota16 = lax.iota(jnp.int32, 16)

        def mkramp(j, _):
            ramp[pl.ds(j * 16, 16)] = iota16 + (segbase + j * 16)
            return 0
        lax.fori_loop(0, NPER // 16, mkramp, 0)

        def dcopy(c, buf):
            rb = pl.multiple_of(row0 + c * _DCH, _DCH)
            return (pltpu.make_async_copy(
                        d_hbm.at[pl.ds(rb, _DCH)], dbuf.at[buf], dsem),)

        def ocopy(bi):
            orb = pl.multiple_of(row0 + bi * _MCH, _MCH)
            return pltpu.make_async_copy(
                ostage.at[bi % 2], gidx_hbm.at[pl.ds(orb, _MCH)], osem)

        for cp in dcopy(0, 0):
            cp.start()

        def do_row(dch, rl, rloc, bi):
            big = jnp.float32(3.2e38)

            for gg in range(8):
                cbuf[pl.ds(gg * 16, 16)] = jnp.full((16,), _INF,
                                                    dtype=jnp.float32)

            def comp(jb, offv):
                for jj in range(8):
                    j = jb * 8 + jj
                    v = dbuf[dch, rl, pl.ds(j * 16, 16)]
                    msk = v < big
                    slot = plsc.bitcast(v, jnp.int32) & 15
                    pos = jnp.minimum(offv + slot, 127)
                    plsc.store_scatter(cbuf, [pos], v, mask=msk)
                    plsc.store_scatter(ibuf, [pos],
                                       ramp[pl.ds(j * 16, 16)], mask=msk)
                    offv = offv + plsc.all_reduce_population_count(msk)
                return offv

            offv = lax.fori_loop(0, (NPER // 16) // 8, comp,
                                 jnp.zeros((16,), jnp.int32))

            g2 = [lax.sort(cbuf[pl.ds(gg * 16, 16)]) for gg in range(8)]
            r0 = _mergek16(g2[0], g2[1])
            r1 = _mergek16(g2[2], g2[3])
            r2 = _mergek16(g2[4], g2[5])
            r3 = _mergek16(g2[6], g2[7])
            w0 = _mergek32(*r0, *r1)
            w1 = _mergek32(*r2, *r3)
            _, s1 = _mergek32(*w0, *w1)
            t32 = jnp.max(s1)

            off2 = jnp.full((16,), -1, jnp.int32)
            for gg in range(8):
                v = cbuf[pl.ds(gg * 16, 16)]
                msk2 = v <= t32
                cs2 = jnp.cumsum(msk2.astype(jnp.int32))
                pos2 = off2 + cs2
                msk3 = msk2 & (pos2 < NS)
                plsc.store_scatter(obuf32, [jnp.minimum(pos2, NS - 1)],
                                   ibuf[pl.ds(gg * 16, 16)], mask=msk3)
                off2 = off2 + plsc.all_reduce_population_count(msk2)
            ostage[bi % 2, rloc, pl.ds(0, 16)] = obuf32[pl.ds(0, 16)]
            ostage[bi % 2, rloc, pl.ds(16, 16)] = obuf32[pl.ds(16, 16)]
            return 0

        def blk16(bi, _):
            @pl.when(bi >= 2)
            def _():
                ocopy(0).wait()
            for rb4 in range(_MCH // _DCH):
                c = bi * (_MCH // _DCH) + rb4
                for cp in dcopy(c, c % 2):
                    cp.wait()

                @pl.when(c + 1 < _RPW1 // _DCH)
                def _():
                    for cp in dcopy(c + 1, (c + 1) % 2):
                        cp.start()
                for rl in range(_DCH):
                    do_row(c % 2, rl, rb4 * _DCH + rl, bi)
            ocopy(bi).start()
            return 0

        lax.fori_loop(0, _RPW1 // _MCH, blk16, 0)
        ocopy(0).wait()
        ocopy(0).wait()

    return body(darr)


_RPW = NE // _NW
_CH_ROWS = 2048
_CH_Q = _CH_ROWS // NS
_NCH = _RPW // _CH_ROWS
_GPC = _CH_ROWS // 128


def _sc_gather_sub(up, vself, idx2):
    mesh = plsc.VectorSubcoreMesh(core_axis_name="c", subcore_axis_name="s")

    @functools.partial(
        pl.kernel,
        out_type=jax.ShapeDtypeStruct((NE, 16), jnp.float32),
        mesh=mesh,
        compiler_params=pltpu.CompilerParams(use_tc_tiling_on_sc=False),
        scratch_types=[
            pltpu.VMEM((_GPC, 128), jnp.int32),
            pltpu.VMEM((_CH_ROWS, 16), jnp.float32),
            pltpu.VMEM((_CH_Q, 16), jnp.float32),
            pltpu.SemaphoreType.DMA,
        ],
    )
    def body(up_hbm, vs_hbm, idx_hbm, out_hbm, idx_v, rows_v, vself_v, sem):
        wid = lax.axis_index("s") * 2 + lax.axis_index("c")
        row_base = wid * _RPW
        q_base = wid * (_RPW // NS)

        def chunk(c, _):
            rb = pl.multiple_of(row_base + c * _CH_ROWS, _CH_ROWS)
            qb = pl.multiple_of(q_base + c * _CH_Q, _CH_Q)
            ib = pl.multiple_of(rb // 128, _GPC)
            pltpu.sync_copy(idx_hbm.at[pl.ds(ib, _GPC)], idx_v)
            cps = [
                pltpu.make_async_copy(
                    up_hbm.at[idx_v.at[j]],
                    rows_v.at[pl.ds(j * 128, 128)],
                    sem,
                )
                for j in range(_GPC)
            ]
            for cp in cps:
                cp.start()
            for cp in cps:
                cp.wait()
            pltpu.sync_copy(vs_hbm.at[pl.ds(qb, _CH_Q)], vself_v)

            def subq(i, _):
                v = vself_v[i]
                base = i * NS
                for s2 in range(NS):
                    rows_v[base + s2] = rows_v[base + s2] - v
                return 0

            lax.fori_loop(0, _CH_Q, subq, 0)
            pltpu.sync_copy(rows_v, out_hbm.at[pl.ds(rb, _CH_ROWS)])
            return 0

        lax.fori_loop(0, _NCH, chunk, 0)

    return body(up, vself, idx2)


_BRF = 8192
_NBF = NE // _BRF
_QF = _BRF // NS


def _fused_body(gu_ref, wl_ref, wf_ref, w1_ref, gl_ref, bl_ref, gf_ref,
                bf_ref, bc1_ref, g1_ref, b1_ref, out_ref,
                macc, mh, sg, tbuf, wbuf):
    p = pl.program_id(0)
    i = pl.program_id(1)
    e = jnp.float32(NE)

    @pl.when((p == 0) & (i == 0))
    def _():
        macc[...] = jnp.zeros_like(macc)

    @pl.when(p == 0)
    def _():
        g = gu_ref[...]
        macc[...] += lax.dot_general(g, g, (((0,), (0,)), ((), ())),
                                     preferred_element_type=jnp.float32)

    @pl.when((p == 1) & (i == 0))
    def _():
        m = macc[...]
        wl = wl_ref[...]
        wf = wf_ref[...]
        mean_gx = m[0:3, 15:16] / e
        cov_gx = m[0:3, 0:3] / e - mean_gx * mean_gx.T
        mu_l = jnp.dot(wl, mean_gx, preferred_element_type=jnp.float32)
        var_l = jnp.sum(jnp.dot(wl, cov_gx,
                                preferred_element_type=jnp.float32) * wl,
                        axis=1, keepdims=True)
        s_l = gl_ref[...] * lax.rsqrt(var_l + EPS)
        mean_p = m[3:3 + CIN, 15:16] / e
        cov_p = m[3:3 + CIN, 3:3 + CIN] / e - mean_p * mean_p.T
        mu_f = jnp.dot(wf, mean_p, preferred_element_type=jnp.float32)
        var_f = jnp.sum(jnp.dot(wf, cov_p,
                                preferred_element_type=jnp.float32) * wf,
                        axis=1, keepdims=True)
        s_f = gf_ref[...] * lax.rsqrt(var_f + EPS)
        tbuf[:, 0:3] = s_l * wl
        tbuf[:, 3:3 + CIN] = s_f * wf
        tbuf[:, 9:15] = jnp.zeros((C0, 6), jnp.float32)
        tbuf[:, 15:16] = (bl_ref[...] - s_l * mu_l
                          + bf_ref[...] - s_f * mu_f)
        mh[...] = jnp.zeros_like(mh)
        sg[...] = jnp.zeros_like(sg)

    @pl.when(p == 1)
    def _():
        g = gu_ref[...]
        t = tbuf[...]
        h = jnp.maximum(
            lax.dot_general(g, t, (((1,), (1,)), ((), ())),
                            preferred_element_type=jnp.float32), 0.0)
        mh[...] += lax.dot_general(h, h, (((0,), (0,)), ((), ())),
                                   preferred_element_type=jnp.float32)
        sg[...] += lax.dot_general(h, g, (((0,), (0,)), ((), ())),
                                   preferred_element_type=jnp.float32)

    @pl.when((p == 2) & (i == 0))
    def _():
        w1 = w1_ref[...]
        bc1 = bc1_ref[...]
        shv = sg[:, 15:16]
        w1sh = jnp.dot(w1, shv, preferred_element_type=jnp.float32) / e
        mean1 = w1sh + bc1
        ey2 = (jnp.sum(jnp.dot(w1, mh[...],
                               preferred_element_type=jnp.float32) * w1,
                       axis=1, keepdims=True) / e
               + 2.0 * bc1 * w1sh + bc1 * bc1)
        var1 = ey2 - mean1 * mean1
        s1 = g1_ref[...] * lax.rsqrt(var1 + EPS)
        wbuf[:, 0:C0] = s1 * w1
        wbuf[:, C0:C0 + 15] = jnp.zeros((C1, 15), jnp.float32)
        wbuf[:, C0 + 15:C0 + 16] = b1_ref[...] + s1 * (bc1 - mean1)

    @pl.when(p == 2)
    def _():
        g = gu_ref[...]
        t = tbuf[...]
        w = wbuf[...]
        h = jnp.maximum(
            lax.dot_general(g, t, (((1,), (1,)), ((), ())),
                            preferred_element_type=jnp.float32), 0.0)
        hg = jnp.concatenate([h, g], axis=1)
        y = jnp.maximum(
            lax.dot_general(hg, w, (((1,), (1,)), ((), ())),
                            preferred_element_type=jnp.float32), 0.0)
        out_ref[...] = jnp.max(y.reshape(_QF, NS, C1), axis=1)


def _fused(gu, wl, wf, w1, gl, bl, gf, bf, bc1, g1, b1):
    small = lambda r, c: pl.BlockSpec((r, c), lambda p, i: (0, 0))
    return pl.pallas_call(
        _fused_body,
        grid=(3, _NBF),
        in_specs=[
            pl.BlockSpec((_BRF, 16), lambda p, i: (i, 0)),
            small(C0, 3), small(C0, CIN), small(C1, C0),
            small(C0, 1), small(C0, 1), small(C0, 1), small(C0, 1),
            small(C1, 1), small(C1, 1), small(C1, 1),
        ],
        out_specs=pl.BlockSpec((_QF, C1), lambda p, i: (i, 0)),
        out_shape=jax.ShapeDtypeStruct((N, C1), jnp.float32),
        scratch_shapes=[
            pltpu.VMEM((16, 16), jnp.float32),
            pltpu.VMEM((C0, C0), jnp.float32),
            pltpu.VMEM((C0, 16), jnp.float32),
            pltpu.VMEM((C0, 16), jnp.float32),
            pltpu.VMEM((C1, C0 + 16), jnp.float32),
        ],
    )(gu, wl, wf, w1, gl, bl, gf, bf, bc1, g1, b1)


def kernel(xyz, points, offset, W_l0, g_l0, b_l0, W_f0, g_f0, b_f0,
           W1, bc1, g1, b1):
    xyzT = xyz.T
    darr, cnt = _dist(xyz, xyzT)
    gidx_fast = _sc_topk(darr)
    gidx = lax.cond(jnp.max(cnt) > 128,
                    lambda: _knn_fallback(xyz, xyzT),
                    lambda: gidx_fast)

    pad = jnp.zeros((N, 16 - 3 - CIN), jnp.float32)
    up = jnp.concatenate(
        [xyz, points, pad[:, :-1], jnp.ones((N, 1), jnp.float32)], axis=1)
    vself = jnp.concatenate([xyz, jnp.zeros((N, 13), jnp.float32)], axis=1)
    idx2 = gidx.reshape(NE // 128, 128)

    gu = _sc_gather_sub(up, vself, idx2)

    new_feats = _fused(
        gu, W_l0, W_f0, W1,
        g_l0.reshape(C0, 1), b_l0.reshape(C0, 1),
        g_f0.reshape(C0, 1), b_f0.reshape(C0, 1),
        bc1.reshape(C1, 1), g1.reshape(C1, 1), b1.reshape(C1, 1))
    return (xyz, new_feats, offset)


# speedup vs baseline: 5.6236x; 1.0277x over previous
import functools

import jax
import jax.numpy as jnp
from jax import lax
from jax.experimental import pallas as pl
from jax.experimental.pallas import tpu as pltpu
from jax.experimental.pallas import tpu_sc as plsc

B = 8
NPER = 2048
N = B * NPER
NS = 32
CIN = 6
C0 = 32
C1 = 64
NE = N * NS
EPS = 1e-5

_BQ = 256
_QB = NPER // _BQ


def _knn_body(q_ref, xt_ref, out_ref):
    seg = pl.program_id(0)
    q = q_ref[...]
    xt = xt_ref[...]
    d = (q[:, 0:1] - xt[0:1, :]) ** 2
    d = d + (q[:, 1:2] - xt[1:2, :]) ** 2
    d = d + (q[:, 2:3] - xt[2:3, :]) ** 2
    fiota = lax.broadcasted_iota(jnp.int32, (_BQ, NPER), 1).astype(jnp.float32)
    kiota = lax.broadcasted_iota(jnp.int32, (_BQ, NS), 1)
    inf = jnp.float32(3.4e38)
    fnper = jnp.float32(NPER)

    def step(k, carry):
        dc, acc = carry
        m = jnp.min(dc, axis=1, keepdims=True)
        sel = jnp.min(jnp.where(dc == m, fiota, fnper), axis=1,
                      keepdims=True)
        acc = jnp.where(kiota == k, sel.astype(jnp.int32), acc)
        dc = jnp.where(fiota == sel, inf, dc)
        return dc, acc

    _, acc = lax.fori_loop(0, NS, step,
                           (d, jnp.zeros((_BQ, NS), jnp.int32)))
    out_ref[...] = acc + seg * NPER


def _knn_fallback(xyz, xyzT):
    return pl.pallas_call(
        _knn_body,
        grid=(B, _QB),
        in_specs=[
            pl.BlockSpec((_BQ, 3), lambda s, q: (s * _QB + q, 0)),
            pl.BlockSpec((3, NPER), lambda s, q: (0, s)),
        ],
        out_specs=pl.BlockSpec((_BQ, NS), lambda s, q: (s * _QB + q, 0)),
        out_shape=jax.ShapeDtypeStruct((N, NS), jnp.int32),
    )(xyz, xyzT)


def _dist_body(q_ref, xt_ref, d_ref, cnt_ref):
    q = q_ref[...]
    xt = xt_ref[...]
    d = (q[:, 0:1] - xt[0:1, :]) ** 2
    d = d + (q[:, 1:2] - xt[1:2, :]) ** 2
    d = d + (q[:, 2:3] - xt[2:3, :]) ** 2
    m = d[:, 0:128]
    for c in range(1, NPER // 128):
        m = jnp.minimum(m, d[:, c * 128:(c + 1) * 128])
    mt = m.T
    inf = jnp.float32(3.4e38)

    def tstep(k, carry):
        cur, _ = carry
        mn = jnp.min(cur, axis=0, keepdims=True)
        cur = jnp.where(cur == mn, inf, cur)
        return cur, mn

    _, t2row = lax.fori_loop(0, NS, tstep, (mt, mt[0:1, :]))
    t2 = t2row.T
    mask = d <= t2
    mi = mask.astype(jnp.int32)
    lanemod = lax.broadcasted_iota(jnp.int32, (_BQ, NPER), 1) % 16
    preb = mask.astype(jnp.bfloat16)
    zb = jnp.zeros((_BQ, 8), jnp.bfloat16)
    for s in (1, 2, 4, 8):
        shifted = jnp.concatenate(
            [zb[:, :s], preb[:, :NPER - s]], axis=1)
        preb = preb + jnp.where(lanemod >= s, shifted, 0)
    pre = preb.astype(jnp.int32)
    du = lax.bitcast_convert_type(d, jnp.int32)
    dpb = (du & jnp.int32(~15)) | (pre - 1)
    d_ref[...] = jnp.where(mask, lax.bitcast_convert_type(dpb, jnp.float32),
                           inf)
    cnt_ref[...] = jnp.broadcast_to(
        jnp.sum(mi, axis=1, keepdims=True), (_BQ, 8))


def _dist(xyz, xyzT):
    return pl.pallas_call(
        _dist_body,
        grid=(B, _QB),
        in_specs=[
            pl.BlockSpec((_BQ, 3), lambda s, q: (s * _QB + q, 0)),
            pl.BlockSpec((3, NPER), lambda s, q: (0, s)),
        ],
        out_specs=[
            pl.BlockSpec((_BQ, NPER), lambda s, q: (s * _QB + q, 0)),
            pl.BlockSpec((_BQ, 8), lambda s, q: (s * _QB + q, 0)),
        ],
        out_shape=[
            jax.ShapeDtypeStruct((N, NPER), jnp.float32),
            jax.ShapeDtypeStruct((N, 8), jnp.int32),
        ],
    )(xyz, xyzT)


_INF = 3.4e38
_NW = 32
_RPW1 = N // _NW
_DCH = 16
_MCH = 16


def _mergek16(a, b):
    rb = lax.rev(b, (0,))
    s = jnp.minimum(a, rb)
    t = jnp.maximum(a, rb)
    return lax.sort(s), lax.sort(t)


def _mergek32(a0, a1, b0, b1):
    rb0 = lax.rev(b0, (0,))
    rb1 = lax.rev(b1, (0,))
    s0 = jnp.minimum(a0, rb1)
    s1 = jnp.minimum(a1, rb0)
    u = jnp.minimum(s0, s1)
    v = jnp.maximum(s0, s1)
    return lax.sort(u), lax.sort(v)


def _sc_topk(darr):
    mesh = plsc.VectorSubcoreMesh(core_axis_name="c", subcore_axis_name="s")

    @functools.partial(
        pl.kernel,
        out_type=jax.ShapeDtypeStruct((N, NS), jnp.int32),
        mesh=mesh,
        compiler_params=pltpu.CompilerParams(use_tc_tiling_on_sc=False,
                                             needs_layout_passes=False),
        scratch_types=[
            pltpu.VMEM((2, _DCH, NPER), jnp.float32),
            pltpu.VMEM((NPER,), jnp.int32),
            pltpu.VMEM((128,), jnp.float32),
            pltpu.VMEM((128,), jnp.int32),
            pltpu.VMEM((2, _MCH, NS), jnp.int32),
            pltpu.VMEM((NS,), jnp.int32),
            pltpu.SemaphoreType.DMA,
            pltpu.SemaphoreType.DMA,
        ],
    )
    def body(d_hbm, gidx_hbm, dbuf, ramp,
             cbuf, ibuf, ostage, obuf32, dsem, osem):
        wid = lax.axis_index("s") * 2 + lax.axis_index("c")
        row0 = wid * _RPW1
        segbase = (row0 // NPER) * NPER
        iota16 = lax.iota(jnp.int32, 16)

        def mkramp(j, _):
            ramp[pl.ds(j * 16, 16)] = iota16 + (segbase + j * 16)
            return 0
        lax.fori_loop(0, NPER // 16, mkramp, 0)

        def dcopy(c, buf):
            rb = pl.multiple_of(row0 + c * _DCH, _DCH)
            return (pltpu.make_async_copy(
                        d_hbm.at[pl.ds(rb, _DCH)], dbuf.at[buf], dsem),)

        def ocopy(bi):
            orb = pl.multiple_of(row0 + bi * _MCH, _MCH)
            return pltpu.make_async_copy(
                ostage.at[bi % 2], gidx_hbm.at[pl.ds(orb, _MCH)], osem)

        for cp in dcopy(0, 0):
            cp.start()

        def do_row(dch, rl, rloc, bi):
            big = jnp.float32(3.2e38)

            for gg in range(8):
                cbuf[pl.ds(gg * 16, 16)] = jnp.full((16,), _INF,
                                                    dtype=jnp.float32)

            def comp(jb, offv):
                for jj in range(8):
                    j = jb * 8 + jj
                    v = dbuf[dch, rl, pl.ds(j * 16, 16)]
                    msk = v < big
                    slot = plsc.bitcast(v, jnp.int32) & 15
                    pos = jnp.minimum(offv + slot, 127)
                    plsc.store_scatter(cbuf, [pos], v, mask=msk)
                    plsc.store_scatter(ibuf, [pos],
                                       ramp[pl.ds(j * 16, 16)], mask=msk)
                    offv = offv + plsc.all_reduce_population_count(msk)
                return offv

            offv = lax.fori_loop(0, (NPER // 16) // 8, comp,
                                 jnp.zeros((16,), jnp.int32))

            g2 = [lax.sort(cbuf[pl.ds(gg * 16, 16)]) for gg in range(8)]
            r0 = _mergek16(g2[0], g2[1])
            r1 = _mergek16(g2[2], g2[3])
            r2 = _mergek16(g2[4], g2[5])
            r3 = _mergek16(g2[6], g2[7])
            w0 = _mergek32(*r0, *r1)
            w1 = _mergek32(*r2, *r3)
            _, s1 = _mergek32(*w0, *w1)
            t32 = jnp.max(s1)

            off2 = jnp.full((16,), -1, jnp.int32)
            for gg in range(8):
                v = cbuf[pl.ds(gg * 16, 16)]
                msk2 = v <= t32
                cs2 = jnp.cumsum(msk2.astype(jnp.int32))
                pos2 = off2 + cs2
                msk3 = msk2 & (pos2 < NS)
                plsc.store_scatter(obuf32, [jnp.minimum(pos2, NS - 1)],
                                   ibuf[pl.ds(gg * 16, 16)], mask=msk3)
                off2 = off2 + plsc.all_reduce_population_count(msk2)
            ostage[bi % 2, rloc, pl.ds(0, 16)] = obuf32[pl.ds(0, 16)]
            ostage[bi % 2, rloc, pl.ds(16, 16)] = obuf32[pl.ds(16, 16)]
            return 0

        def blk16(bi, _):
            @pl.when(bi >= 2)
            def _():
                ocopy(0).wait()
            for rb4 in range(_MCH // _DCH):
                c = bi * (_MCH // _DCH) + rb4
                for cp in dcopy(c, c % 2):
                    cp.wait()

                @pl.when(c + 1 < _RPW1 // _DCH)
                def _():
                    for cp in dcopy(c + 1, (c + 1) % 2):
                        cp.start()
                for rl in range(_DCH):
                    do_row(c % 2, rl, rb4 * _DCH + rl, bi)
            ocopy(bi).start()
            return 0

        lax.fori_loop(0, _RPW1 // _MCH, blk16, 0)
        ocopy(0).wait()
        ocopy(0).wait()

    return body(darr)


_RPW = NE // _NW
_CH_ROWS = 2048
_CH_Q = _CH_ROWS // NS
_NCH = _RPW // _CH_ROWS
_GPC = _CH_ROWS // 128


def _sc_gather_sub(up, vself, idx2):
    mesh = plsc.VectorSubcoreMesh(core_axis_name="c", subcore_axis_name="s")

    @functools.partial(
        pl.kernel,
        out_type=jax.ShapeDtypeStruct((NE, 16), jnp.float32),
        mesh=mesh,
        compiler_params=pltpu.CompilerParams(use_tc_tiling_on_sc=False),
        scratch_types=[
            pltpu.VMEM((_GPC, 128), jnp.int32),
            pltpu.VMEM((_CH_ROWS, 16), jnp.float32),
            pltpu.VMEM((_CH_Q, 16), jnp.float32),
            pltpu.SemaphoreType.DMA,
        ],
    )
    def body(up_hbm, vs_hbm, idx_hbm, out_hbm, idx_v, rows_v, vself_v, sem):
        wid = lax.axis_index("s") * 2 + lax.axis_index("c")
        row_base = wid * _RPW
        q_base = wid * (_RPW // NS)

        def chunk(c, _):
            rb = pl.multiple_of(row_base + c * _CH_ROWS, _CH_ROWS)
            qb = pl.multiple_of(q_base + c * _CH_Q, _CH_Q)
            ib = pl.multiple_of(rb // 128, _GPC)
            pltpu.sync_copy(idx_hbm.at[pl.ds(ib, _GPC)], idx_v)
            cps = [
                pltpu.make_async_copy(
                    up_hbm.at[idx_v.at[j]],
                    rows_v.at[pl.ds(j * 128, 128)],
                    sem,
                )
                for j in range(_GPC)
            ]
            for cp in cps:
                cp.start()
            for cp in cps:
                cp.wait()
            pltpu.sync_copy(vs_hbm.at[pl.ds(qb, _CH_Q)], vself_v)

            def subq(i, _):
                v = vself_v[i]
                base = i * NS
                for s2 in range(NS):
                    rows_v[base + s2] = rows_v[base + s2] - v
                return 0

            lax.fori_loop(0, _CH_Q, subq, 0)
            pltpu.sync_copy(rows_v, out_hbm.at[pl.ds(rb, _CH_ROWS)])
            return 0

        lax.fori_loop(0, _NCH, chunk, 0)

    return body(up, vself, idx2)


_BRF = 8192
_NBF = NE // _BRF
_QF = _BRF // NS


def _fused_body(gu_ref, wl_ref, wf_ref, w1_ref, gl_ref, bl_ref, gf_ref,
                bf_ref, bc1_ref, g1_ref, b1_ref, out_ref,
                macc, mh, sg, tbuf, wbuf):
    p = pl.program_id(0)
    i = pl.program_id(1)
    e = jnp.float32(NE)

    @pl.when((p == 0) & (i == 0))
    def _():
        macc[...] = jnp.zeros_like(macc)

    @pl.when(p == 0)
    def _():
        g = gu_ref[...]
        macc[...] += lax.dot_general(g, g, (((0,), (0,)), ((), ())),
                                     preferred_element_type=jnp.float32)

    @pl.when((p == 1) & (i == 0))
    def _():
        m = macc[...]
        wl = wl_ref[...]
        wf = wf_ref[...]
        mean_gx = m[0:3, 15:16] / e
        cov_gx = m[0:3, 0:3] / e - mean_gx * mean_gx.T
        mu_l = jnp.dot(wl, mean_gx, preferred_element_type=jnp.float32)
        var_l = jnp.sum(jnp.dot(wl, cov_gx,
                                preferred_element_type=jnp.float32) * wl,
                        axis=1, keepdims=True)
        s_l = gl_ref[...] * lax.rsqrt(var_l + EPS)
        mean_p = m[3:3 + CIN, 15:16] / e
        cov_p = m[3:3 + CIN, 3:3 + CIN] / e - mean_p * mean_p.T
        mu_f = jnp.dot(wf, mean_p, preferred_element_type=jnp.float32)
        var_f = jnp.sum(jnp.dot(wf, cov_p,
                                preferred_element_type=jnp.float32) * wf,
                        axis=1, keepdims=True)
        s_f = gf_ref[...] * lax.rsqrt(var_f + EPS)
        tbuf[:, 0:3] = s_l * wl
        tbuf[:, 3:3 + CIN] = s_f * wf
        tbuf[:, 9:15] = jnp.zeros((C0, 6), jnp.float32)
        tbuf[:, 15:16] = (bl_ref[...] - s_l * mu_l
                          + bf_ref[...] - s_f * mu_f)
        mh[...] = jnp.zeros_like(mh)
        sg[...] = jnp.zeros_like(sg)

    @pl.when(p == 1)
    def _():
        g = gu_ref[...]
        t = tbuf[...]
        h = jnp.maximum(
            lax.dot_general(g, t, (((1,), (1,)), ((), ())),
                            preferred_element_type=jnp.float32), 0.0)
        mh[...] += lax.dot_general(h, h, (((0,), (0,)), ((), ())),
                                   preferred_element_type=jnp.float32)
        sg[...] += lax.dot_general(h, g, (((0,), (0,)), ((), ())),
                                   preferred_element_type=jnp.float32)

    @pl.when((p == 2) & (i == 0))
    def _():
        w1 = w1_ref[...]
        bc1 = bc1_ref[...]
        shv = sg[:, 15:16]
        w1sh = jnp.dot(w1, shv, preferred_element_type=jnp.float32) / e
        mean1 = w1sh + bc1
        ey2 = (jnp.sum(jnp.dot(w1, mh[...],
                               preferred_element_type=jnp.float32) * w1,
                       axis=1, keepdims=True) / e
               + 2.0 * bc1 * w1sh + bc1 * bc1)
        var1 = ey2 - mean1 * mean1
        s1 = g1_ref[...] * lax.rsqrt(var1 + EPS)
        wbuf[:, 0:C0] = s1 * w1
        wbuf[:, C0:C0 + 15] = jnp.zeros((C1, 15), jnp.float32)
        wbuf[:, C0 + 15:C0 + 16] = b1_ref[...] + s1 * (bc1 - mean1)

    @pl.when(p == 2)
    def _():
        g = gu_ref[...]
        t = tbuf[...]
        w = wbuf[...]
        h = jnp.maximum(
            lax.dot_general(g, t, (((1,), (1,)), ((), ())),
                            preferred_element_type=jnp.float32), 0.0)
        hg = jnp.concatenate([h, g], axis=1)
        y = jnp.maximum(
            lax.dot_general(hg, w, (((1,), (1,)), ((), ())),
                            preferred_element_type=jnp.float32), 0.0)
        out_ref[...] = jnp.max(y.reshape(_QF, NS, C1), axis=1)


def _fused(gu, wl, wf, w1, gl, bl, gf, bf, bc1, g1, b1):
    small = lambda r, c: pl.BlockSpec((r, c), lambda p, i: (0, 0))
    return pl.pallas_call(
        _fused_body,
        grid=(3, _NBF),
        in_specs=[
            pl.BlockSpec((_BRF, 16), lambda p, i: (i, 0)),
            small(C0, 3), small(C0, CIN), small(C1, C0),
            small(C0, 1), small(C0, 1), small(C0, 1), small(C0, 1),
            small(C1, 1), small(C1, 1), small(C1, 1),
        ],
        out_specs=pl.BlockSpec((_QF, C1), lambda p, i: (i, 0)),
        out_shape=jax.ShapeDtypeStruct((N, C1), jnp.float32),
        scratch_shapes=[
            pltpu.VMEM((16, 16), jnp.float32),
            pltpu.VMEM((C0, C0), jnp.float32),
            pltpu.VMEM((C0, 16), jnp.float32),
            pltpu.VMEM((C0, 16), jnp.float32),
            pltpu.VMEM((C1, C0 + 16), jnp.float32),
        ],
    )(gu, wl, wf, w1, gl, bl, gf, bf, bc1, g1, b1)


def kernel(xyz, points, offset, W_l0, g_l0, b_l0, W_f0, g_f0, b_f0,
           W1, bc1, g1, b1):
    xyzT = xyz.T
    darr, cnt = _dist(xyz, xyzT)
    gidx_fast = _sc_topk(darr)
    gidx = lax.cond(jnp.max(cnt) > 128,
                    lambda: _knn_fallback(xyz, xyzT),
                    lambda: gidx_fast)

    pad = jnp.zeros((N, 16 - 3 - CIN), jnp.float32)
    up = jnp.concatenate(
        [xyz, points, pad[:, :-1], jnp.ones((N, 1), jnp.float32)], axis=1)
    vself = jnp.concatenate([xyz, jnp.zeros((N, 13), jnp.float32)], axis=1)
    idx2 = gidx.reshape(NE // 128, 128)

    gu = _sc_gather_sub(up, vself, idx2)

    new_feats = _fused(
        gu, W_l0, W_f0, W1,
        g_l0.reshape(C0, 1), b_l0.reshape(C0, 1),
        g_f0.reshape(C0, 1), b_f0.reshape(C0, 1),
        bc1.reshape(C1, 1), g1.reshape(C1, 1), b1.reshape(C1, 1))
    return (xyz, new_feats, offset)
